# baseline (speedup 1.0000x reference)
import functools

import jax
import jax.numpy as jnp
from jax import lax
from jax.experimental import pallas as pl
from jax.experimental.pallas import tpu as pltpu
from jax.experimental.pallas import tpu_sc as plsc

N = 10000
NP = 10240
E = 320000
EPAD = 327680
H = 128
G = 128
F_ATOM = 9
NC, NS, LANES = 2, 16, 16
NW = NC * NS
EPT = EPAD // NW
EC = 32
NCHUNK_E = EPT // EC
NPT = NP // NW
AC = 80
NCHUNK_A = NPT // AC
STRIPE = NP // NS
BR = 512
NB = NP // BR

def _vec_loop_add_relu(dst, src, rows):
    def _row(r, carry):
        for c in range(H // LANES):
            sl = pl.ds(c * LANES, LANES)
            dst[r, sl] = jnp.maximum(dst[r, sl] + src[r, sl], 0.0)
        return carry
    lax.fori_loop(0, rows, _row, 0)


def _vec_loop_add(dst, src, rows):
    def _row(r, carry):
        for c in range(H // LANES):
            sl = pl.ds(c * LANES, LANES)
            dst[r, sl] = dst[r, sl] + src[r, sl]
        return carry
    lax.fori_loop(0, rows, _row, 0)


def _atom_body(atab_hbm, paix_hbm, out_hbm, pidxa, acc, tmp, asem):
    cid = lax.axis_index("c")
    sid = lax.axis_index("s")
    wid = sid * NC + cid
    g0 = wid * NCHUNK_A

    def _chunk(k, carry):
        pltpu.sync_copy(paix_hbm.at[g0 + k], pidxa)
        for f in range(F_ATOM):
            pltpu.async_copy(atab_hbm.at[pidxa.at[f]], tmp.at[f], asem)
        for f in range(F_ATOM):
            pltpu.make_async_copy(
                atab_hbm.at[pl.ds(0, AC)], tmp.at[f], asem).wait()

        def _row(r, c2):
            for c in range(H // LANES):
                sl = pl.ds(c * LANES, LANES)
                v = tmp[0, r, sl]
                for f in range(1, F_ATOM):
                    v = v + tmp[f, r, sl]
                acc[r, sl] = v
            return c2

        lax.fori_loop(0, AC, _row, 0)
        pltpu.sync_copy(acc, out_hbm.at[pl.ds(wid * NPT + k * AC, AC)])
        return carry

    lax.fori_loop(0, NCHUNK_A, _chunk, 0)


def _edge_body(h_hbm, etab_hbm, pidx_hbm, zrows_hbm, out_hbm,
               pidx, dstc, hs, es, ms, agg_sh,
               isem0, isem1, isem2, isem3, gsem0, gsem1, ssem0, ssem1):
    cid = lax.axis_index("c")
    sid = lax.axis_index("s")
    wid = sid * NC + cid
    isems = (isem0, isem1, isem2, isem3)
    gsems, ssems = (gsem0, gsem1), (ssem0, ssem1)

    def issue_idx(g, q):
        pltpu.async_copy(pidx_hbm.at[g], pidx.at[q], isems[q])

    def wait_idx(q):
        pltpu.make_async_copy(pidx_hbm.at[0], pidx.at[q], isems[q]).wait()

    def issue_gathers(b, q):
        pltpu.async_copy(h_hbm.at[pidx.at[q, 0]], hs.at[b], gsems[b])
        pltpu.async_copy(etab_hbm.at[pidx.at[q, 1]], es.at[b], gsems[b])

    def wait_gathers(b):
        pltpu.make_async_copy(h_hbm.at[pl.ds(0, EC)], hs.at[b], gsems[b]).wait()
        pltpu.make_async_copy(etab_hbm.at[pl.ds(0, EC)], es.at[b], gsems[b]).wait()

    def copy_dst(b, q):
        for c in range(EC // LANES):
            sl = pl.ds(c * LANES, LANES)
            dstc[b, sl] = pidx[q, 2, sl]

    def compute(b):
        def _row(r, carry):
            for c in range(H // LANES):
                sl = pl.ds(c * LANES, LANES)
                ms[b, r, sl] = jnp.maximum(hs[b, r, sl] + es[b, r, sl], 0.0)
            return carry
        lax.fori_loop(0, EC, _row, 0)

    def issue_scatter(b):
        pltpu.async_copy(ms.at[b], agg_sh.at[dstc.at[b]], ssems[b], add=True)

    def wait_scatter(b):
        pltpu.make_async_copy(ms.at[b], agg_sh.at[pl.ds(0, EC)], ssems[b]).wait()

    pltpu.sync_copy(zrows_hbm.at[pl.ds(0, EC)], ms.at[0])
    for k in range(STRIPE // EC):
        pltpu.sync_copy(ms.at[0], agg_sh.at[pl.ds(sid * STRIPE + k * EC, EC)])
    plsc.subcore_barrier()

    base = wid * NCHUNK_E
    for q in range(4):
        issue_idx(base + q, q)
    wait_idx(0)
    issue_gathers(0, 0)
    wait_idx(1)
    issue_gathers(1, 1)

    def _step(j, b, q, sw, pf_idx, pf_g):
        wait_gathers(b)
        if sw:
            wait_scatter(b)
        copy_dst(b, q)
        if pf_idx:
            issue_idx(base + j + 4, q)
        compute(b)
        issue_scatter(b)
        if pf_g:
            wait_idx((q + 2) % 4)
            issue_gathers(b, (q + 2) % 4)

    _step(0, 0, 0, False, True, True)
    _step(1, 1, 1, False, True, True)

    def _mid(it, carry):
        for d in range(4):
            j = 2 + it * 4 + d
            _step(j, (2 + d) % 2, (2 + d) % 4, True, True, True)
        return carry

    lax.fori_loop(0, (NCHUNK_E - 8) // 4, _mid, 0)
    for j in range(NCHUNK_E - 6, NCHUNK_E):
        _step(j, j % 2, j % 4,
              True, j < NCHUNK_E - 4, j < NCHUNK_E - 2)
    wait_scatter(0)
    wait_scatter(1)
    plsc.subcore_barrier()

    for k in range(STRIPE // EC):
        b = k % 2
        r0 = sid * STRIPE + k * EC
        if k >= 2:
            pltpu.make_async_copy(
                ms.at[b], out_hbm.at[pl.ds(0, EC)], ssems[b]).wait()
        pltpu.sync_copy(agg_sh.at[pl.ds(r0, EC)], ms.at[b])
        pltpu.async_copy(ms.at[b], out_hbm.at[pl.ds(cid * NP + r0, EC)],
                         ssems[b])
    for b in range(2):
        pltpu.make_async_copy(
            ms.at[b], out_hbm.at[pl.ds(0, EC)], ssems[b]).wait()


@functools.cache
def _sc_kernels():
    mesh = plsc.VectorSubcoreMesh(core_axis_name="c", subcore_axis_name="s")
    atom = pl.kernel(
        _atom_body,
        out_type=jax.ShapeDtypeStruct((NP, H), jnp.float32),
        mesh=mesh,
        scratch_types=[
            pltpu.VMEM((F_ATOM, AC), jnp.int32),
            pltpu.VMEM((AC, H), jnp.float32),
            pltpu.VMEM((F_ATOM, AC, H), jnp.float32),
            pltpu.SemaphoreType.DMA,
        ],
    )
    edge = pl.kernel(
        _edge_body,
        out_type=jax.ShapeDtypeStruct((2 * NP, H), jnp.float32),
        mesh=mesh,
        scratch_types=[
            pltpu.VMEM((4, 3, EC), jnp.int32),
            pltpu.VMEM((2, EC), jnp.int32),
            pltpu.VMEM((2, EC, H), jnp.float32),
            pltpu.VMEM((2, EC, H), jnp.float32),
            pltpu.VMEM((2, EC, H), jnp.float32),
            pltpu.VMEM_SHARED((NP, H), jnp.float32),
        ] + [pltpu.SemaphoreType.DMA] * 8,
    )
    return atom, edge


def _indicator(b):
    return (b[:, None] == lax.broadcasted_iota(jnp.int32, (1, G), 1)
            ).astype(jnp.float32)


def _post_body(one_eps, h_in, agg_a, agg_b, vn, batch3,
               w1p, b1p, w2, b2, v1a, v1b, b1v, h_out, pooled):
    he = h_in[...]
    z = one_eps[0, 0] * he + (agg_a[...] + agg_b[...])
    z = jnp.maximum(
        jnp.dot(z, w1p[...], preferred_element_type=jnp.float32) + b1p[...], 0.0)
    hn = jnp.maximum(
        jnp.dot(z, w2[...], preferred_element_type=jnp.float32) + b2[...], 0.0)
    h_out[...] = hn
    ind = _indicator(batch3[0, 0, :])
    vnb = jnp.dot(ind, vn[...], preferred_element_type=jnp.float32)
    vt = jnp.maximum(
        jnp.dot(vnb, v1a[...], preferred_element_type=jnp.float32)
        + jnp.dot(hn, v1b[...], preferred_element_type=jnp.float32)
        + b1v[...], 0.0)
    part = lax.dot_general(ind, vt, (((0,), (0,)), ((), ())),
                           preferred_element_type=jnp.float32)

    @pl.when(pl.program_id(0) == 0)
    def _():
        pooled[...] = jnp.zeros_like(pooled)

    pooled[...] += part


def _post_tc(one_eps, h_in, aggs, vn, batch3, w1p, b1p, w2, b2, v1a, v1b, b1v):
    full = lambda shape: pl.BlockSpec(shape, lambda i: (0,) * len(shape))
    return pl.pallas_call(
        _post_body,
        grid=(NB,),
        in_specs=[
            full((1, 1)),
            pl.BlockSpec((BR, H), lambda i: (i, 0)),
            pl.BlockSpec((BR, H), lambda i: (i, 0)),
            pl.BlockSpec((BR, H), lambda i: (i + NB, 0)),
            full((G, H)),
            pl.BlockSpec((1, 1, BR), lambda i: (i, 0, 0)),
            full((H, 2 * H)), full((1, 2 * H)),
            full((2 * H, H)), full((1, H)),
            full((H, 2 * H)), full((H, 2 * H)), full((1, 2 * H)),
        ],
        out_specs=[
            pl.BlockSpec((BR, H), lambda i: (i, 0)),
            pl.BlockSpec((G, 2 * H), lambda i: (0, 0)),
        ],
        out_shape=[
            jax.ShapeDtypeStruct((NP, H), jnp.float32),
            jax.ShapeDtypeStruct((G, 2 * H), jnp.float32),
        ],
    )(one_eps, h_in, aggs, aggs, vn, batch3, w1p, b1p, w2, b2, v1a, v1b, b1v)


def _pre_body(pooled, h, batch3, v2w, v2b, h_in_out, vn_out):
    vn = jnp.maximum(
        jnp.dot(pooled[...], v2w[...], preferred_element_type=jnp.float32)
        + v2b[...], 0.0)
    ind = _indicator(batch3[0, 0, :])
    h_in_out[...] = h[...] + jnp.dot(ind, vn, preferred_element_type=jnp.float32)

    @pl.when(pl.program_id(0) == 0)
    def _():
        vn_out[...] = vn


def _pre_tc(pooled, h, batch3, v2w, v2b):
    full = lambda shape: pl.BlockSpec(shape, lambda i: (0,) * len(shape))
    return pl.pallas_call(
        _pre_body,
        grid=(NB,),
        in_specs=[
            full((G, 2 * H)),
            pl.BlockSpec((BR, H), lambda i: (i, 0)),
            pl.BlockSpec((1, 1, BR), lambda i: (i, 0, 0)),
            full((2 * H, H)), full((1, H)),
        ],
        out_specs=[
            pl.BlockSpec((BR, H), lambda i: (i, 0)),
            pl.BlockSpec((G, H), lambda i: (0, 0)),
        ],
        out_shape=[
            jax.ShapeDtypeStruct((NP, H), jnp.float32),
            jax.ShapeDtypeStruct((G, H), jnp.float32),
        ],
    )(pooled, h, batch3, v2w, v2b)


def _final_body(pooled, v2w, v2b, pw1, pb1, pw2, pb2, out):
    vn = jnp.maximum(
        jnp.dot(pooled[...], v2w[...], preferred_element_type=jnp.float32)
        + v2b[...], 0.0)
    p1 = jnp.maximum(
        jnp.dot(vn, pw1[...], preferred_element_type=jnp.float32) + pb1[...], 0.0)
    p2 = jnp.dot(p1, pw2[...], preferred_element_type=jnp.float32) + pb2[...]
    out[...] = jnp.clip(p2, 0.0, 50.0)


def _final_tc(pooled, v2w, v2b, pw1, pb1, pw2, pb2):
    return pl.pallas_call(
        _final_body,
        out_shape=jax.ShapeDtypeStruct((G, 1), jnp.float32),
    )(pooled, v2w, v2b, pw1, pb1, pw2, pb2)


def _fold_bn(w, b, g, bb):
    return w * g[None, :], (b * g + bb)[None, :]


def kernel(x, edge_index, edge_attr, batch, params):
    f32, i32 = jnp.float32, jnp.int32
    xi = x.astype(i32) + 100 * jnp.arange(F_ATOM, dtype=i32)[None, :]
    xi = jnp.pad(xi, ((0, NP - N), (0, 0)))
    paix = (xi.T.reshape(F_ATOM, NW, NCHUNK_A, AC)
            .transpose(1, 2, 0, 3).reshape(NW * NCHUNK_A, F_ATOM, AC))
    atab = params['atom_emb'].reshape(F_ATOM * 100, H)

    src = jnp.concatenate(
        [edge_index[0].astype(i32), jnp.zeros((EPAD - E,), i32)])
    dst = jnp.concatenate(
        [edge_index[1].astype(i32), jnp.full((EPAD - E,), NP - 1, i32)])
    ea3 = edge_attr.astype(i32)
    ea = jnp.concatenate(
        [ea3[:, 0] * 100 + ea3[:, 1] * 10 + ea3[:, 2],
         jnp.zeros((EPAD - E,), i32)])
    pidx = jnp.stack([a.reshape(NW * NCHUNK_E, EC) for a in (src, ea, dst)],
                     axis=1)
    zrows = jnp.zeros((STRIPE, H), f32)

    batch_p = jnp.concatenate(
        [batch.astype(i32), jnp.full((NP - N,), G, i32)])
    batch3 = batch_p.reshape(NB, 1, BR)

    atom_sc, edge_sc = _sc_kernels()
    h0 = atom_sc(atab, paix)

    layers = params['layers']
    etab_s, v2w_s, v2b_s = [], [], []
    w1p_s, b1p_s, w2_s, b2_s, v1a_s, v1b_s, b1v_s, eps_s = ([] for _ in range(8))
    for l, lp in enumerate(layers):
        etab_s.append((lp['bond_emb'][0][:, None, None, :]
                       + lp['bond_emb'][1][None, :, None, :]
                       + lp['bond_emb'][2][None, None, :, :]).reshape(1000, H))
        if l == 0:
            v2w_s.append(jnp.zeros((2 * H, H), f32))
            v2b_s.append(jnp.zeros((1, H), f32))
        else:
            pp = layers[l - 1]
            vw, vb = _fold_bn(pp['vn2_w'], pp['vn2_b'], pp['vn2_g'], pp['vn2_bb'])
            v2w_s.append(vw)
            v2b_s.append(vb)
        w1p, b1p = _fold_bn(lp['w1'], lp['b1'], lp['bn1_g'], lp['bn1_b'])
        v1w, b1v = _fold_bn(lp['vn1_w'], lp['vn1_b'], lp['vn1_g'], lp['vn1_bb'])
        w1p_s.append(w1p); b1p_s.append(b1p)
        w2_s.append(lp['w2']); b2_s.append(lp['b2'][None, :])
        v1a_s.append(v1w[:H]); v1b_s.append(v1w[H:]); b1v_s.append(b1v)
        eps_s.append((1.0 + lp['eps']).astype(f32).reshape(1, 1))
    xs = tuple(jnp.stack(a) for a in (
        etab_s, v2w_s, v2b_s, w1p_s, b1p_s, w2_s, b2_s, v1a_s, v1b_s,
        b1v_s, eps_s))

    def layer_step(carry, x):
        h_prev, pooled = carry
        etab, v2w, v2b, w1p, b1p, w2, b2, v1a, v1b, b1v, one_eps = x
        h_in, vn = _pre_tc(pooled, h_prev, batch3, v2w, v2b)
        aggs = edge_sc(h_in, etab, pidx, zrows)
        h_new, pooled_new = _post_tc(
            one_eps, h_in, aggs, vn, batch3, w1p, b1p, w2, b2, v1a, v1b, b1v)
        return (h_new, pooled_new), None

    (_, pooled), _ = lax.scan(
        layer_step, (h0, jnp.zeros((G, 2 * H), f32)), xs)

    lp_last = layers[-1]
    v2w, v2b = _fold_bn(lp_last['vn2_w'], lp_last['vn2_b'],
                        lp_last['vn2_g'], lp_last['vn2_bb'])
    return _final_tc(pooled, v2w, v2b,
                     params['pred_w1'], params['pred_b1'][None, :],
                     params['pred_w2'], params['pred_b2'][None, :])


# speedup vs baseline: 1.1604x; 1.1604x over previous
import functools

import jax
import jax.numpy as jnp
from jax import lax
from jax.experimental import pallas as pl
from jax.experimental.pallas import tpu as pltpu
from jax.experimental.pallas import tpu_sc as plsc

N = 10000
NP = 10240
E = 320000
EPAD = 327680
H = 128
G = 128
F_ATOM = 9
NC, NS, LANES = 2, 16, 16
NW = NC * NS
EPT = EPAD // NW
EC = 80
GCH = EPAD // EC
KCH0 = 172
KCH1 = 84
NPT = NP // NW
AC = 80
NCHUNK_A = NPT // AC
STRIPE = NP // NS
BR = 512
NB = NP // BR

def _vec_loop_add_relu(dst, src, rows):
    def _row(r, carry):
        for c in range(H // LANES):
            sl = pl.ds(c * LANES, LANES)
            dst[r, sl] = jnp.maximum(dst[r, sl] + src[r, sl], 0.0)
        return carry
    lax.fori_loop(0, rows, _row, 0)


def _vec_loop_add(dst, src, rows):
    def _row(r, carry):
        for c in range(H // LANES):
            sl = pl.ds(c * LANES, LANES)
            dst[r, sl] = dst[r, sl] + src[r, sl]
        return carry
    lax.fori_loop(0, rows, _row, 0)


def _atom_body(atab_hbm, paix_hbm, out_hbm, pidxa, acc, tmp, asem):
    cid = lax.axis_index("c")
    sid = lax.axis_index("s")
    wid = sid * NC + cid
    g0 = wid * NCHUNK_A

    def _chunk(k, carry):
        pltpu.sync_copy(paix_hbm.at[g0 + k], pidxa)
        for f in range(F_ATOM):
            pltpu.async_copy(atab_hbm.at[pidxa.at[f]], tmp.at[f], asem)
        for f in range(F_ATOM):
            pltpu.make_async_copy(
                atab_hbm.at[pl.ds(0, AC)], tmp.at[f], asem).wait()

        def _row(r, c2):
            for c in range(H // LANES):
                sl = pl.ds(c * LANES, LANES)
                v = tmp[0, r, sl]
                for f in range(1, F_ATOM):
                    v = v + tmp[f, r, sl]
                acc[r, sl] = v
            return c2

        lax.fori_loop(0, AC, _row, 0)
        pltpu.sync_copy(acc, out_hbm.at[pl.ds(wid * NPT + k * AC, AC)])
        return carry

    lax.fori_loop(0, NCHUNK_A, _chunk, 0)


def _edge_body(h_hbm, etab_hbm, pidx_hbm, zrows_hbm, out_hbm,
               pidx, dstc, hs, es, agg_sh,
               isem0, isem1, isem2, isem3, gsem0, gsem1, ssem0, ssem1):
    cid = lax.axis_index("c")
    sid = lax.axis_index("s")
    isems = (isem0, isem1, isem2, isem3)
    gsems, ssems = (gsem0, gsem1), (ssem0, ssem1)

    def issue_idx(g, q):
        pltpu.async_copy(pidx_hbm.at[g], pidx.at[q], isems[q])

    def wait_idx(q):
        pltpu.make_async_copy(pidx_hbm.at[0], pidx.at[q], isems[q]).wait()

    def issue_gathers(b, q):
        pltpu.async_copy(h_hbm.at[pidx.at[q, 0]], hs.at[b], gsems[b])
        pltpu.async_copy(etab_hbm.at[pidx.at[q, 1]], es.at[b], gsems[b])

    def wait_gathers(b):
        pltpu.make_async_copy(h_hbm.at[pl.ds(0, EC)], hs.at[b], gsems[b]).wait()
        pltpu.make_async_copy(etab_hbm.at[pl.ds(0, EC)], es.at[b], gsems[b]).wait()

    def copy_dst(b, q):
        for c in range(EC // LANES):
            sl = pl.ds(c * LANES, LANES)
            dstc[b, sl] = pidx[q, 2, sl]

    def compute(b):
        def _row(r, carry):
            for c in range(H // LANES):
                sl = pl.ds(c * LANES, LANES)
                hs[b, r, sl] = jnp.maximum(hs[b, r, sl] + es[b, r, sl], 0.0)
            return carry
        lax.fori_loop(0, EC, _row, 0)

    def issue_scatter(b):
        pltpu.async_copy(hs.at[b], agg_sh.at[dstc.at[b]], ssems[b], add=True)

    def wait_scatter(b):
        pltpu.make_async_copy(hs.at[b], agg_sh.at[pl.ds(0, EC)], ssems[b]).wait()

    pltpu.sync_copy(zrows_hbm.at[pl.ds(0, EC)], hs.at[0])
    for k in range(STRIPE // EC):
        pltpu.sync_copy(hs.at[0], agg_sh.at[pl.ds(sid * STRIPE + k * EC, EC)])
    plsc.subcore_barrier()

    def _step(j, b, q, pf_idx, pf_g):
        wait_gathers(b)
        copy_dst(b, q)
        if pf_idx:
            issue_idx(cbase + j + 4, q)
        compute(b)
        issue_scatter(b)
        if pf_g:
            wait_idx((q + 2) % 4)
            wait_scatter(b)
            issue_gathers(b, (q + 2) % 4)

    def run_edges(nch):
        for q in range(4):
            issue_idx(cbase + q, q)
        wait_idx(0)
        issue_gathers(0, 0)
        wait_idx(1)
        issue_gathers(1, 1)
        _step(0, 0, 0, True, True)
        _step(1, 1, 1, True, True)

        def _mid(it, carry):
            for d in range(4):
                j = 2 + it * 4 + d
                _step(j, (2 + d) % 2, (2 + d) % 4, True, True)
            return carry

        lax.fori_loop(0, (nch - 8) // 4, _mid, 0)
        for j in range(nch - 6, nch):
            _step(j, j % 2, j % 4, j < nch - 4, j < nch - 2)
        wait_scatter(0)
        wait_scatter(1)

    cbase = jnp.where(cid == 0, sid * KCH0, 16 * KCH0 + sid * KCH1)

    @pl.when(cid == 0)
    def _():
        run_edges(KCH0)

    @pl.when(cid == 1)
    def _():
        run_edges(KCH1)

    plsc.subcore_barrier()

    for k in range(STRIPE // EC):
        b = k % 2
        r0 = sid * STRIPE + k * EC
        if k >= 2:
            pltpu.make_async_copy(
                hs.at[b], out_hbm.at[pl.ds(0, EC)], ssems[b]).wait()
        pltpu.sync_copy(agg_sh.at[pl.ds(r0, EC)], hs.at[b])
        pltpu.async_copy(hs.at[b], out_hbm.at[pl.ds(cid * NP + r0, EC)],
                         ssems[b])
    for b in range(2):
        pltpu.make_async_copy(
            hs.at[b], out_hbm.at[pl.ds(0, EC)], ssems[b]).wait()


@functools.cache
def _sc_kernels():
    mesh = plsc.VectorSubcoreMesh(core_axis_name="c", subcore_axis_name="s")
    atom = pl.kernel(
        _atom_body,
        out_type=jax.ShapeDtypeStruct((NP, H), jnp.float32),
        mesh=mesh,
        scratch_types=[
            pltpu.VMEM((F_ATOM, AC), jnp.int32),
            pltpu.VMEM((AC, H), jnp.float32),
            pltpu.VMEM((F_ATOM, AC, H), jnp.float32),
            pltpu.SemaphoreType.DMA,
        ],
    )
    edge = pl.kernel(
        _edge_body,
        out_type=jax.ShapeDtypeStruct((2 * NP, H), jnp.float32),
        mesh=mesh,
        scratch_types=[
            pltpu.VMEM((4, 3, EC), jnp.int32),
            pltpu.VMEM((2, EC), jnp.int32),
            pltpu.VMEM((2, EC, H), jnp.float32),
            pltpu.VMEM((2, EC, H), jnp.float32),
            pltpu.VMEM_SHARED((NP, H), jnp.float32),
        ] + [pltpu.SemaphoreType.DMA] * 8,
    )
    return atom, edge


def _indicator(b):
    return (b[:, None] == lax.broadcasted_iota(jnp.int32, (1, G), 1)
            ).astype(jnp.float32)


def _post_body(one_eps, h_in, agg_a, agg_b, vn, batch3,
               w1p, b1p, w2, b2, v1a, v1b, b1v, h_out, pooled):
    he = h_in[...]
    z = one_eps[0, 0] * he + (agg_a[...] + agg_b[...])
    z = jnp.maximum(
        jnp.dot(z, w1p[...], preferred_element_type=jnp.float32) + b1p[...], 0.0)
    hn = jnp.maximum(
        jnp.dot(z, w2[...], preferred_element_type=jnp.float32) + b2[...], 0.0)
    h_out[...] = hn
    ind = _indicator(batch3[0, 0, :])
    vnb = jnp.dot(ind, vn[...], preferred_element_type=jnp.float32)
    vt = jnp.maximum(
        jnp.dot(vnb, v1a[...], preferred_element_type=jnp.float32)
        + jnp.dot(hn, v1b[...], preferred_element_type=jnp.float32)
        + b1v[...], 0.0)
    part = lax.dot_general(ind, vt, (((0,), (0,)), ((), ())),
                           preferred_element_type=jnp.float32)

    @pl.when(pl.program_id(0) == 0)
    def _():
        pooled[...] = jnp.zeros_like(pooled)

    pooled[...] += part


def _post_tc(one_eps, h_in, aggs, vn, batch3, w1p, b1p, w2, b2, v1a, v1b, b1v):
    full = lambda shape: pl.BlockSpec(shape, lambda i: (0,) * len(shape))
    return pl.pallas_call(
        _post_body,
        grid=(NB,),
        in_specs=[
            full((1, 1)),
            pl.BlockSpec((BR, H), lambda i: (i, 0)),
            pl.BlockSpec((BR, H), lambda i: (i, 0)),
            pl.BlockSpec((BR, H), lambda i: (i + NB, 0)),
            full((G, H)),
            pl.BlockSpec((1, 1, BR), lambda i: (i, 0, 0)),
            full((H, 2 * H)), full((1, 2 * H)),
            full((2 * H, H)), full((1, H)),
            full((H, 2 * H)), full((H, 2 * H)), full((1, 2 * H)),
        ],
        out_specs=[
            pl.BlockSpec((BR, H), lambda i: (i, 0)),
            pl.BlockSpec((G, 2 * H), lambda i: (0, 0)),
        ],
        out_shape=[
            jax.ShapeDtypeStruct((NP, H), jnp.float32),
            jax.ShapeDtypeStruct((G, 2 * H), jnp.float32),
        ],
    )(one_eps, h_in, aggs, aggs, vn, batch3, w1p, b1p, w2, b2, v1a, v1b, b1v)


def _pre_body(pooled, h, batch3, v2w, v2b, h_in_out, vn_out):
    vn = jnp.maximum(
        jnp.dot(pooled[...], v2w[...], preferred_element_type=jnp.float32)
        + v2b[...], 0.0)
    ind = _indicator(batch3[0, 0, :])
    h_in_out[...] = h[...] + jnp.dot(ind, vn, preferred_element_type=jnp.float32)

    @pl.when(pl.program_id(0) == 0)
    def _():
        vn_out[...] = vn


def _pre_tc(pooled, h, batch3, v2w, v2b):
    full = lambda shape: pl.BlockSpec(shape, lambda i: (0,) * len(shape))
    return pl.pallas_call(
        _pre_body,
        grid=(NB,),
        in_specs=[
            full((G, 2 * H)),
            pl.BlockSpec((BR, H), lambda i: (i, 0)),
            pl.BlockSpec((1, 1, BR), lambda i: (i, 0, 0)),
            full((2 * H, H)), full((1, H)),
        ],
        out_specs=[
            pl.BlockSpec((BR, H), lambda i: (i, 0)),
            pl.BlockSpec((G, H), lambda i: (0, 0)),
        ],
        out_shape=[
            jax.ShapeDtypeStruct((NP, H), jnp.float32),
            jax.ShapeDtypeStruct((G, H), jnp.float32),
        ],
    )(pooled, h, batch3, v2w, v2b)


def _final_body(pooled, v2w, v2b, pw1, pb1, pw2, pb2, out):
    vn = jnp.maximum(
        jnp.dot(pooled[...], v2w[...], preferred_element_type=jnp.float32)
        + v2b[...], 0.0)
    p1 = jnp.maximum(
        jnp.dot(vn, pw1[...], preferred_element_type=jnp.float32) + pb1[...], 0.0)
    p2 = jnp.dot(p1, pw2[...], preferred_element_type=jnp.float32) + pb2[...]
    out[...] = jnp.clip(p2, 0.0, 50.0)


def _final_tc(pooled, v2w, v2b, pw1, pb1, pw2, pb2):
    return pl.pallas_call(
        _final_body,
        out_shape=jax.ShapeDtypeStruct((G, 1), jnp.float32),
    )(pooled, v2w, v2b, pw1, pb1, pw2, pb2)


def _fold_bn(w, b, g, bb):
    return w * g[None, :], (b * g + bb)[None, :]


def kernel(x, edge_index, edge_attr, batch, params):
    f32, i32 = jnp.float32, jnp.int32
    xi = x.astype(i32) + 100 * jnp.arange(F_ATOM, dtype=i32)[None, :]
    xi = jnp.pad(xi, ((0, NP - N), (0, 0)))
    paix = (xi.T.reshape(F_ATOM, NW, NCHUNK_A, AC)
            .transpose(1, 2, 0, 3).reshape(NW * NCHUNK_A, F_ATOM, AC))
    atab = params['atom_emb'].reshape(F_ATOM * 100, H)

    src = jnp.concatenate(
        [edge_index[0].astype(i32), jnp.zeros((EPAD - E,), i32)])
    dst = jnp.concatenate(
        [edge_index[1].astype(i32), jnp.full((EPAD - E,), NP - 1, i32)])
    ea3 = edge_attr.astype(i32)
    ea = jnp.concatenate(
        [ea3[:, 0] * 100 + ea3[:, 1] * 10 + ea3[:, 2],
         jnp.zeros((EPAD - E,), i32)])
    pidx = jnp.stack([a.reshape(GCH, EC) for a in (src, ea, dst)], axis=1)
    zrows = jnp.zeros((STRIPE, H), f32)

    batch_p = jnp.concatenate(
        [batch.astype(i32), jnp.full((NP - N,), G, i32)])
    batch3 = batch_p.reshape(NB, 1, BR)

    atom_sc, edge_sc = _sc_kernels()
    h0 = atom_sc(atab, paix)

    layers = params['layers']
    etab_s, v2w_s, v2b_s = [], [], []
    w1p_s, b1p_s, w2_s, b2_s, v1a_s, v1b_s, b1v_s, eps_s = ([] for _ in range(8))
    for l, lp in enumerate(layers):
        etab_s.append((lp['bond_emb'][0][:, None, None, :]
                       + lp['bond_emb'][1][None, :, None, :]
                       + lp['bond_emb'][2][None, None, :, :]).reshape(1000, H))
        if l == 0:
            v2w_s.append(jnp.zeros((2 * H, H), f32))
            v2b_s.append(jnp.zeros((1, H), f32))
        else:
            pp = layers[l - 1]
            vw, vb = _fold_bn(pp['vn2_w'], pp['vn2_b'], pp['vn2_g'], pp['vn2_bb'])
            v2w_s.append(vw)
            v2b_s.append(vb)
        w1p, b1p = _fold_bn(lp['w1'], lp['b1'], lp['bn1_g'], lp['bn1_b'])
        v1w, b1v = _fold_bn(lp['vn1_w'], lp['vn1_b'], lp['vn1_g'], lp['vn1_bb'])
        w1p_s.append(w1p); b1p_s.append(b1p)
        w2_s.append(lp['w2']); b2_s.append(lp['b2'][None, :])
        v1a_s.append(v1w[:H]); v1b_s.append(v1w[H:]); b1v_s.append(b1v)
        eps_s.append((1.0 + lp['eps']).astype(f32).reshape(1, 1))
    xs = tuple(jnp.stack(a) for a in (
        etab_s, v2w_s, v2b_s, w1p_s, b1p_s, w2_s, b2_s, v1a_s, v1b_s,
        b1v_s, eps_s))

    def layer_step(carry, x):
        h_prev, pooled = carry
        etab, v2w, v2b, w1p, b1p, w2, b2, v1a, v1b, b1v, one_eps = x
        h_in, vn = _pre_tc(pooled, h_prev, batch3, v2w, v2b)
        aggs = edge_sc(h_in, etab, pidx, zrows)
        h_new, pooled_new = _post_tc(
            one_eps, h_in, aggs, vn, batch3, w1p, b1p, w2, b2, v1a, v1b, b1v)
        return (h_new, pooled_new), None

    (_, pooled), _ = lax.scan(
        layer_step, (h0, jnp.zeros((G, 2 * H), f32)), xs)

    lp_last = layers[-1]
    v2w, v2b = _fold_bn(lp_last['vn2_w'], lp_last['vn2_b'],
                        lp_last['vn2_g'], lp_last['vn2_bb'])
    return _final_tc(pooled, v2w, v2b,
                     params['pred_w1'], params['pred_b1'][None, :],
                     params['pred_w2'], params['pred_b2'][None, :])


# speedup vs baseline: 1.1693x; 1.0076x over previous
import functools

import jax
import jax.numpy as jnp
from jax import lax
from jax.experimental import pallas as pl
from jax.experimental.pallas import tpu as pltpu
from jax.experimental.pallas import tpu_sc as plsc

N = 10000
NP = 10240
E = 320000
EPAD = 327680
H = 128
G = 128
F_ATOM = 9
NC, NS, LANES = 2, 16, 16
NW = NC * NS
EPT = EPAD // NW
EC = 80
GCH = EPAD // EC
KCH0 = 200
KCH1 = 56
NPT = NP // NW
AC = 80
NCHUNK_A = NPT // AC
STRIPE = NP // NS
BR = 512
NB = NP // BR

def _vec_loop_add_relu(dst, src, rows):
    def _row(r, carry):
        for c in range(H // LANES):
            sl = pl.ds(c * LANES, LANES)
            dst[r, sl] = jnp.maximum(dst[r, sl] + src[r, sl], 0.0)
        return carry
    lax.fori_loop(0, rows, _row, 0)


def _vec_loop_add(dst, src, rows):
    def _row(r, carry):
        for c in range(H // LANES):
            sl = pl.ds(c * LANES, LANES)
            dst[r, sl] = dst[r, sl] + src[r, sl]
        return carry
    lax.fori_loop(0, rows, _row, 0)


def _atom_body(atab_hbm, paix_hbm, out_hbm, pidxa, acc, tmp, asem):
    cid = lax.axis_index("c")
    sid = lax.axis_index("s")
    wid = sid * NC + cid
    g0 = wid * NCHUNK_A

    def _chunk(k, carry):
        pltpu.sync_copy(paix_hbm.at[g0 + k], pidxa)
        for f in range(F_ATOM):
            pltpu.async_copy(atab_hbm.at[pidxa.at[f]], tmp.at[f], asem)
        for f in range(F_ATOM):
            pltpu.make_async_copy(
                atab_hbm.at[pl.ds(0, AC)], tmp.at[f], asem).wait()

        def _row(r, c2):
            for c in range(H // LANES):
                sl = pl.ds(c * LANES, LANES)
                v = tmp[0, r, sl]
                for f in range(1, F_ATOM):
                    v = v + tmp[f, r, sl]
                acc[r, sl] = v
            return c2

        lax.fori_loop(0, AC, _row, 0)
        pltpu.sync_copy(acc, out_hbm.at[pl.ds(wid * NPT + k * AC, AC)])
        return carry

    lax.fori_loop(0, NCHUNK_A, _chunk, 0)


def _edge_body(h_hbm, etab_hbm, pidx_hbm, zrows_hbm, out_hbm,
               pidx, dstc, hs, es, agg_sh,
               isem0, isem1, isem2, isem3, gsem0, gsem1, ssem0, ssem1):
    cid = lax.axis_index("c")
    sid = lax.axis_index("s")
    isems = (isem0, isem1, isem2, isem3)
    gsems, ssems = (gsem0, gsem1), (ssem0, ssem1)

    def issue_idx(g, q):
        pltpu.async_copy(pidx_hbm.at[g], pidx.at[q], isems[q])

    def wait_idx(q):
        pltpu.make_async_copy(pidx_hbm.at[0], pidx.at[q], isems[q]).wait()

    def issue_gathers(b, q):
        pltpu.async_copy(h_hbm.at[pidx.at[q, 0]], hs.at[b], gsems[b])
        pltpu.async_copy(etab_hbm.at[pidx.at[q, 1]], es.at[b], gsems[b])

    def wait_gathers(b):
        pltpu.make_async_copy(h_hbm.at[pl.ds(0, EC)], hs.at[b], gsems[b]).wait()
        pltpu.make_async_copy(etab_hbm.at[pl.ds(0, EC)], es.at[b], gsems[b]).wait()

    def copy_dst(b, q):
        for c in range(EC // LANES):
            sl = pl.ds(c * LANES, LANES)
            dstc[b, sl] = pidx[q, 2, sl]

    def compute(b):
        def _row(r, carry):
            for c in range(H // LANES):
                sl = pl.ds(c * LANES, LANES)
                hs[b, r, sl] = jnp.maximum(hs[b, r, sl] + es[b, r, sl], 0.0)
            return carry
        lax.fori_loop(0, EC, _row, 0)

    def issue_scatter(b):
        pltpu.async_copy(hs.at[b], agg_sh.at[dstc.at[b]], ssems[b], add=True)

    def wait_scatter(b):
        pltpu.make_async_copy(hs.at[b], agg_sh.at[pl.ds(0, EC)], ssems[b]).wait()

    pltpu.sync_copy(zrows_hbm.at[pl.ds(0, EC)], hs.at[0])
    for k in range(STRIPE // EC):
        pltpu.sync_copy(hs.at[0], agg_sh.at[pl.ds(sid * STRIPE + k * EC, EC)])
    plsc.subcore_barrier()

    def _step(j, b, q, pf_idx, pf_g):
        wait_gathers(b)
        copy_dst(b, q)
        if pf_idx:
            issue_idx(cbase + j + 4, q)
        compute(b)
        issue_scatter(b)
        if pf_g:
            wait_idx((q + 2) % 4)
            wait_scatter(b)
            issue_gathers(b, (q + 2) % 4)

    def run_edges(nch):
        for q in range(4):
            issue_idx(cbase + q, q)
        wait_idx(0)
        issue_gathers(0, 0)
        wait_idx(1)
        issue_gathers(1, 1)
        _step(0, 0, 0, True, True)
        _step(1, 1, 1, True, True)

        def _mid(it, carry):
            for d in range(4):
                j = 2 + it * 4 + d
                _step(j, (2 + d) % 2, (2 + d) % 4, True, True)
            return carry

        lax.fori_loop(0, (nch - 8) // 4, _mid, 0)
        for j in range(nch - 6, nch):
            _step(j, j % 2, j % 4, j < nch - 4, j < nch - 2)
        wait_scatter(0)
        wait_scatter(1)

    cbase = jnp.where(cid == 0, sid * KCH0, 16 * KCH0 + sid * KCH1)

    @pl.when(cid == 0)
    def _():
        run_edges(KCH0)

    @pl.when(cid == 1)
    def _():
        run_edges(KCH1)

    plsc.subcore_barrier()

    for k in range(STRIPE // EC):
        b = k % 2
        r0 = sid * STRIPE + k * EC
        if k >= 2:
            pltpu.make_async_copy(
                hs.at[b], out_hbm.at[pl.ds(0, EC)], ssems[b]).wait()
        pltpu.sync_copy(agg_sh.at[pl.ds(r0, EC)], hs.at[b])
        pltpu.async_copy(hs.at[b], out_hbm.at[pl.ds(cid * NP + r0, EC)],
                         ssems[b])
    for b in range(2):
        pltpu.make_async_copy(
            hs.at[b], out_hbm.at[pl.ds(0, EC)], ssems[b]).wait()


@functools.cache
def _sc_kernels():
    mesh = plsc.VectorSubcoreMesh(core_axis_name="c", subcore_axis_name="s")
    atom = pl.kernel(
        _atom_body,
        out_type=jax.ShapeDtypeStruct((NP, H), jnp.float32),
        mesh=mesh,
        scratch_types=[
            pltpu.VMEM((F_ATOM, AC), jnp.int32),
            pltpu.VMEM((AC, H), jnp.float32),
            pltpu.VMEM((F_ATOM, AC, H), jnp.float32),
            pltpu.SemaphoreType.DMA,
        ],
    )
    edge = pl.kernel(
        _edge_body,
        out_type=jax.ShapeDtypeStruct((2 * NP, H), jnp.float32),
        mesh=mesh,
        scratch_types=[
            pltpu.VMEM((4, 3, EC), jnp.int32),
            pltpu.VMEM((2, EC), jnp.int32),
            pltpu.VMEM((2, EC, H), jnp.float32),
            pltpu.VMEM((2, EC, H), jnp.float32),
            pltpu.VMEM_SHARED((NP, H), jnp.float32),
        ] + [pltpu.SemaphoreType.DMA] * 8,
    )
    return atom, edge


def _indicator(b):
    return (b[:, None] == lax.broadcasted_iota(jnp.int32, (1, G), 1)
            ).astype(jnp.float32)


def _post_body(one_eps, h_in, agg_a, agg_b, vn, batch3,
               w1p, b1p, w2, b2, v1a, v1b, b1v, h_out, pooled):
    he = h_in[...]
    z = one_eps[0, 0] * he + (agg_a[...] + agg_b[...])
    z = jnp.maximum(
        jnp.dot(z, w1p[...], preferred_element_type=jnp.float32) + b1p[...], 0.0)
    hn = jnp.maximum(
        jnp.dot(z, w2[...], preferred_element_type=jnp.float32) + b2[...], 0.0)
    h_out[...] = hn
    ind = _indicator(batch3[0, 0, :])
    vnb = jnp.dot(ind, vn[...], preferred_element_type=jnp.float32)
    vt = jnp.maximum(
        jnp.dot(vnb, v1a[...], preferred_element_type=jnp.float32)
        + jnp.dot(hn, v1b[...], preferred_element_type=jnp.float32)
        + b1v[...], 0.0)
    part = lax.dot_general(ind, vt, (((0,), (0,)), ((), ())),
                           preferred_element_type=jnp.float32)

    @pl.when(pl.program_id(0) == 0)
    def _():
        pooled[...] = jnp.zeros_like(pooled)

    pooled[...] += part


def _post_tc(one_eps, h_in, aggs, vn, batch3, w1p, b1p, w2, b2, v1a, v1b, b1v):
    full = lambda shape: pl.BlockSpec(shape, lambda i: (0,) * len(shape))
    return pl.pallas_call(
        _post_body,
        grid=(NB,),
        in_specs=[
            full((1, 1)),
            pl.BlockSpec((BR, H), lambda i: (i, 0)),
            pl.BlockSpec((BR, H), lambda i: (i, 0)),
            pl.BlockSpec((BR, H), lambda i: (i + NB, 0)),
            full((G, H)),
            pl.BlockSpec((1, 1, BR), lambda i: (i, 0, 0)),
            full((H, 2 * H)), full((1, 2 * H)),
            full((2 * H, H)), full((1, H)),
            full((H, 2 * H)), full((H, 2 * H)), full((1, 2 * H)),
        ],
        out_specs=[
            pl.BlockSpec((BR, H), lambda i: (i, 0)),
            pl.BlockSpec((G, 2 * H), lambda i: (0, 0)),
        ],
        out_shape=[
            jax.ShapeDtypeStruct((NP, H), jnp.float32),
            jax.ShapeDtypeStruct((G, 2 * H), jnp.float32),
        ],
    )(one_eps, h_in, aggs, aggs, vn, batch3, w1p, b1p, w2, b2, v1a, v1b, b1v)


def _pre_body(pooled, h, batch3, v2w, v2b, h_in_out, vn_out):
    vn = jnp.maximum(
        jnp.dot(pooled[...], v2w[...], preferred_element_type=jnp.float32)
        + v2b[...], 0.0)
    ind = _indicator(batch3[0, 0, :])
    h_in_out[...] = h[...] + jnp.dot(ind, vn, preferred_element_type=jnp.float32)

    @pl.when(pl.program_id(0) == 0)
    def _():
        vn_out[...] = vn


def _pre_tc(pooled, h, batch3, v2w, v2b):
    full = lambda shape: pl.BlockSpec(shape, lambda i: (0,) * len(shape))
    return pl.pallas_call(
        _pre_body,
        grid=(NB,),
        in_specs=[
            full((G, 2 * H)),
            pl.BlockSpec((BR, H), lambda i: (i, 0)),
            pl.BlockSpec((1, 1, BR), lambda i: (i, 0, 0)),
            full((2 * H, H)), full((1, H)),
        ],
        out_specs=[
            pl.BlockSpec((BR, H), lambda i: (i, 0)),
            pl.BlockSpec((G, H), lambda i: (0, 0)),
        ],
        out_shape=[
            jax.ShapeDtypeStruct((NP, H), jnp.float32),
            jax.ShapeDtypeStruct((G, H), jnp.float32),
        ],
    )(pooled, h, batch3, v2w, v2b)


def _final_body(pooled, v2w, v2b, pw1, pb1, pw2, pb2, out):
    vn = jnp.maximum(
        jnp.dot(pooled[...], v2w[...], preferred_element_type=jnp.float32)
        + v2b[...], 0.0)
    p1 = jnp.maximum(
        jnp.dot(vn, pw1[...], preferred_element_type=jnp.float32) + pb1[...], 0.0)
    p2 = jnp.dot(p1, pw2[...], preferred_element_type=jnp.float32) + pb2[...]
    out[...] = jnp.clip(p2, 0.0, 50.0)


def _final_tc(pooled, v2w, v2b, pw1, pb1, pw2, pb2):
    return pl.pallas_call(
        _final_body,
        out_shape=jax.ShapeDtypeStruct((G, 1), jnp.float32),
    )(pooled, v2w, v2b, pw1, pb1, pw2, pb2)


def _fold_bn(w, b, g, bb):
    return w * g[None, :], (b * g + bb)[None, :]


def kernel(x, edge_index, edge_attr, batch, params):
    f32, i32 = jnp.float32, jnp.int32
    xi = x.astype(i32) + 100 * jnp.arange(F_ATOM, dtype=i32)[None, :]
    xi = jnp.pad(xi, ((0, NP - N), (0, 0)))
    paix = (xi.T.reshape(F_ATOM, NW, NCHUNK_A, AC)
            .transpose(1, 2, 0, 3).reshape(NW * NCHUNK_A, F_ATOM, AC))
    atab = params['atom_emb'].reshape(F_ATOM * 100, H)

    src = jnp.concatenate(
        [edge_index[0].astype(i32), jnp.zeros((EPAD - E,), i32)])
    dst = jnp.concatenate(
        [edge_index[1].astype(i32), jnp.full((EPAD - E,), NP - 1, i32)])
    ea3 = edge_attr.astype(i32)
    ea = jnp.concatenate(
        [ea3[:, 0] * 100 + ea3[:, 1] * 10 + ea3[:, 2],
         jnp.zeros((EPAD - E,), i32)])
    pidx = jnp.stack([a.reshape(GCH, EC) for a in (src, ea, dst)], axis=1)
    zrows = jnp.zeros((STRIPE, H), f32)

    batch_p = jnp.concatenate(
        [batch.astype(i32), jnp.full((NP - N,), G, i32)])
    batch3 = batch_p.reshape(NB, 1, BR)

    atom_sc, edge_sc = _sc_kernels()
    h0 = atom_sc(atab, paix)

    layers = params['layers']
    etab_s, v2w_s, v2b_s = [], [], []
    w1p_s, b1p_s, w2_s, b2_s, v1a_s, v1b_s, b1v_s, eps_s = ([] for _ in range(8))
    for l, lp in enumerate(layers):
        etab_s.append((lp['bond_emb'][0][:, None, None, :]
                       + lp['bond_emb'][1][None, :, None, :]
                       + lp['bond_emb'][2][None, None, :, :]).reshape(1000, H))
        if l == 0:
            v2w_s.append(jnp.zeros((2 * H, H), f32))
            v2b_s.append(jnp.zeros((1, H), f32))
        else:
            pp = layers[l - 1]
            vw, vb = _fold_bn(pp['vn2_w'], pp['vn2_b'], pp['vn2_g'], pp['vn2_bb'])
            v2w_s.append(vw)
            v2b_s.append(vb)
        w1p, b1p = _fold_bn(lp['w1'], lp['b1'], lp['bn1_g'], lp['bn1_b'])
        v1w, b1v = _fold_bn(lp['vn1_w'], lp['vn1_b'], lp['vn1_g'], lp['vn1_bb'])
        w1p_s.append(w1p); b1p_s.append(b1p)
        w2_s.append(lp['w2']); b2_s.append(lp['b2'][None, :])
        v1a_s.append(v1w[:H]); v1b_s.append(v1w[H:]); b1v_s.append(b1v)
        eps_s.append((1.0 + lp['eps']).astype(f32).reshape(1, 1))
    xs = tuple(jnp.stack(a) for a in (
        etab_s, v2w_s, v2b_s, w1p_s, b1p_s, w2_s, b2_s, v1a_s, v1b_s,
        b1v_s, eps_s))

    def layer_step(carry, x):
        h_prev, pooled = carry
        etab, v2w, v2b, w1p, b1p, w2, b2, v1a, v1b, b1v, one_eps = x
        h_in, vn = _pre_tc(pooled, h_prev, batch3, v2w, v2b)
        aggs = edge_sc(h_in, etab, pidx, zrows)
        h_new, pooled_new = _post_tc(
            one_eps, h_in, aggs, vn, batch3, w1p, b1p, w2, b2, v1a, v1b, b1v)
        return (h_new, pooled_new), None

    (_, pooled), _ = lax.scan(
        layer_step, (h0, jnp.zeros((G, 2 * H), f32)), xs)

    lp_last = layers[-1]
    v2w, v2b = _fold_bn(lp_last['vn2_w'], lp_last['vn2_b'],
                        lp_last['vn2_g'], lp_last['vn2_bb'])
    return _final_tc(pooled, v2w, v2b,
                     params['pred_w1'], params['pred_b1'][None, :],
                     params['pred_w2'], params['pred_b2'][None, :])


# speedup vs baseline: 1.1791x; 1.0084x over previous
import functools

import jax
import jax.numpy as jnp
from jax import lax
from jax.experimental import pallas as pl
from jax.experimental.pallas import tpu as pltpu
from jax.experimental.pallas import tpu_sc as plsc

N = 10000
NP = 10240
E = 320000
EPAD = 327680
H = 128
G = 128
F_ATOM = 9
NC, NS, LANES = 2, 16, 16
NW = NC * NS
EPT = EPAD // NW
EC = 80
GCH = EPAD // EC
KCH0 = 200
KCH1 = 56
NPT = NP // NW
AC = 80
NCHUNK_A = NPT // AC
STRIPE = NP // NS
BR = 512
NB = NP // BR

def _vec_loop_add_relu(dst, src, rows):
    def _row(r, carry):
        for c in range(H // LANES):
            sl = pl.ds(c * LANES, LANES)
            dst[r, sl] = jnp.maximum(dst[r, sl] + src[r, sl], 0.0)
        return carry
    lax.fori_loop(0, rows, _row, 0)


def _vec_loop_add(dst, src, rows):
    def _row(r, carry):
        for c in range(H // LANES):
            sl = pl.ds(c * LANES, LANES)
            dst[r, sl] = dst[r, sl] + src[r, sl]
        return carry
    lax.fori_loop(0, rows, _row, 0)


def _atom_body(atab_hbm, paix_hbm, out_hbm, pidxa, acc, tmp, asem):
    cid = lax.axis_index("c")
    sid = lax.axis_index("s")

    def _chunk(g0):
        def body(k, carry):
            g = g0 + k
            pltpu.sync_copy(paix_hbm.at[g], pidxa)
            for f in range(F_ATOM):
                pltpu.async_copy(atab_hbm.at[pidxa.at[f]], tmp.at[f], asem)
            for f in range(F_ATOM):
                pltpu.make_async_copy(
                    atab_hbm.at[pl.ds(0, AC)], tmp.at[f], asem).wait()

            def _row(r, c2):
                for c in range(H // LANES):
                    sl = pl.ds(c * LANES, LANES)
                    v = tmp[0, r, sl]
                    for f in range(1, F_ATOM):
                        v = v + tmp[f, r, sl]
                    acc[r, sl] = v
                return c2

            lax.fori_loop(0, AC, _row, 0)
            pltpu.sync_copy(acc, out_hbm.at[pl.ds(g * AC, AC)])
            return carry
        return body

    @pl.when(cid == 0)
    def _():
        lax.fori_loop(0, 6, _chunk(sid * 6), 0)

    @pl.when(cid == 1)
    def _():
        lax.fori_loop(0, 2, _chunk(16 * 6 + sid * 2), 0)


def _edge_body(h_hbm, etab_hbm, pidx_hbm, zrows_hbm, out_hbm,
               pidx, dstc, hs, es, agg_sh,
               isem0, isem1, isem2, isem3, gsem0, gsem1, ssem0, ssem1):
    cid = lax.axis_index("c")
    sid = lax.axis_index("s")
    isems = (isem0, isem1, isem2, isem3)
    gsems, ssems = (gsem0, gsem1), (ssem0, ssem1)

    def issue_idx(g, q):
        pltpu.async_copy(pidx_hbm.at[g], pidx.at[q], isems[q])

    def wait_idx(q):
        pltpu.make_async_copy(pidx_hbm.at[0], pidx.at[q], isems[q]).wait()

    def issue_gathers(b, q):
        pltpu.async_copy(h_hbm.at[pidx.at[q, 0]], hs.at[b], gsems[b])
        pltpu.async_copy(etab_hbm.at[pidx.at[q, 1]], es.at[b], gsems[b])

    def wait_gathers(b):
        pltpu.make_async_copy(h_hbm.at[pl.ds(0, EC)], hs.at[b], gsems[b]).wait()
        pltpu.make_async_copy(etab_hbm.at[pl.ds(0, EC)], es.at[b], gsems[b]).wait()

    def copy_dst(b, q):
        for c in range(EC // LANES):
            sl = pl.ds(c * LANES, LANES)
            dstc[b, sl] = pidx[q, 2, sl]

    def compute(b):
        def _row(r, carry):
            for c in range(H // LANES):
                sl = pl.ds(c * LANES, LANES)
                hs[b, r, sl] = jnp.maximum(hs[b, r, sl] + es[b, r, sl], 0.0)
            return carry
        lax.fori_loop(0, EC, _row, 0)

    def issue_scatter(b):
        pltpu.async_copy(hs.at[b], agg_sh.at[dstc.at[b]], ssems[b], add=True)

    def wait_scatter(b):
        pltpu.make_async_copy(hs.at[b], agg_sh.at[pl.ds(0, EC)], ssems[b]).wait()

    pltpu.sync_copy(zrows_hbm.at[pl.ds(0, EC)], hs.at[0])
    for k in range(STRIPE // EC):
        pltpu.sync_copy(hs.at[0], agg_sh.at[pl.ds(sid * STRIPE + k * EC, EC)])
    plsc.subcore_barrier()

    def _step(j, b, q, pf_idx, pf_g):
        wait_gathers(b)
        copy_dst(b, q)
        if pf_idx:
            issue_idx(cbase + j + 4, q)
        compute(b)
        issue_scatter(b)
        if pf_g:
            wait_idx((q + 2) % 4)
            wait_scatter(b)
            issue_gathers(b, (q + 2) % 4)

    def run_edges(nch):
        for q in range(4):
            issue_idx(cbase + q, q)
        wait_idx(0)
        issue_gathers(0, 0)
        wait_idx(1)
        issue_gathers(1, 1)
        _step(0, 0, 0, True, True)
        _step(1, 1, 1, True, True)

        def _mid(it, carry):
            for d in range(4):
                j = 2 + it * 4 + d
                _step(j, (2 + d) % 2, (2 + d) % 4, True, True)
            return carry

        lax.fori_loop(0, (nch - 8) // 4, _mid, 0)
        for j in range(nch - 6, nch):
            _step(j, j % 2, j % 4, j < nch - 4, j < nch - 2)
        wait_scatter(0)
        wait_scatter(1)

    cbase = jnp.where(cid == 0, sid * KCH0, 16 * KCH0 + sid * KCH1)

    @pl.when(cid == 0)
    def _():
        run_edges(KCH0)

    @pl.when(cid == 1)
    def _():
        run_edges(KCH1)

    plsc.subcore_barrier()

    for k in range(STRIPE // EC):
        b = k % 2
        r0 = sid * STRIPE + k * EC
        if k >= 2:
            pltpu.make_async_copy(
                hs.at[b], out_hbm.at[pl.ds(0, EC)], ssems[b]).wait()
        pltpu.sync_copy(agg_sh.at[pl.ds(r0, EC)], hs.at[b])
        pltpu.async_copy(hs.at[b], out_hbm.at[pl.ds(cid * NP + r0, EC)],
                         ssems[b])
    for b in range(2):
        pltpu.make_async_copy(
            hs.at[b], out_hbm.at[pl.ds(0, EC)], ssems[b]).wait()


@functools.cache
def _sc_kernels():
    mesh = plsc.VectorSubcoreMesh(core_axis_name="c", subcore_axis_name="s")
    atom = pl.kernel(
        _atom_body,
        out_type=jax.ShapeDtypeStruct((NP, H), jnp.float32),
        mesh=mesh,
        scratch_types=[
            pltpu.VMEM((F_ATOM, AC), jnp.int32),
            pltpu.VMEM((AC, H), jnp.float32),
            pltpu.VMEM((F_ATOM, AC, H), jnp.float32),
            pltpu.SemaphoreType.DMA,
        ],
    )
    edge = pl.kernel(
        _edge_body,
        out_type=jax.ShapeDtypeStruct((2 * NP, H), jnp.float32),
        mesh=mesh,
        scratch_types=[
            pltpu.VMEM((4, 3, EC), jnp.int32),
            pltpu.VMEM((2, EC), jnp.int32),
            pltpu.VMEM((2, EC, H), jnp.float32),
            pltpu.VMEM((2, EC, H), jnp.float32),
            pltpu.VMEM_SHARED((NP, H), jnp.float32),
        ] + [pltpu.SemaphoreType.DMA] * 8,
    )
    return atom, edge


def _indicator(b):
    return (b[:, None] == lax.broadcasted_iota(jnp.int32, (1, G), 1)
            ).astype(jnp.float32)


def _post_body(one_eps, h_in, agg_a, agg_b, vn, batch3,
               w1p, b1p, w2, b2, v1a, v1b, b1v, h_out, pooled):
    he = h_in[...]
    z = one_eps[0, 0] * he + (agg_a[...] + agg_b[...])
    z = jnp.maximum(
        jnp.dot(z, w1p[...], preferred_element_type=jnp.float32) + b1p[...], 0.0)
    hn = jnp.maximum(
        jnp.dot(z, w2[...], preferred_element_type=jnp.float32) + b2[...], 0.0)
    h_out[...] = hn
    ind = _indicator(batch3[0, 0, :])
    vnb = jnp.dot(ind, vn[...], preferred_element_type=jnp.float32)
    vt = jnp.maximum(
        jnp.dot(vnb, v1a[...], preferred_element_type=jnp.float32)
        + jnp.dot(hn, v1b[...], preferred_element_type=jnp.float32)
        + b1v[...], 0.0)
    part = lax.dot_general(ind, vt, (((0,), (0,)), ((), ())),
                           preferred_element_type=jnp.float32)

    @pl.when(pl.program_id(0) == 0)
    def _():
        pooled[...] = jnp.zeros_like(pooled)

    pooled[...] += part


def _post_tc(one_eps, h_in, aggs, vn, batch3, w1p, b1p, w2, b2, v1a, v1b, b1v):
    full = lambda shape: pl.BlockSpec(shape, lambda i: (0,) * len(shape))
    return pl.pallas_call(
        _post_body,
        grid=(NB,),
        in_specs=[
            full((1, 1)),
            pl.BlockSpec((BR, H), lambda i: (i, 0)),
            pl.BlockSpec((BR, H), lambda i: (i, 0)),
            pl.BlockSpec((BR, H), lambda i: (i + NB, 0)),
            full((G, H)),
            pl.BlockSpec((1, 1, BR), lambda i: (i, 0, 0)),
            full((H, 2 * H)), full((1, 2 * H)),
            full((2 * H, H)), full((1, H)),
            full((H, 2 * H)), full((H, 2 * H)), full((1, 2 * H)),
        ],
        out_specs=[
            pl.BlockSpec((BR, H), lambda i: (i, 0)),
            pl.BlockSpec((G, 2 * H), lambda i: (0, 0)),
        ],
        out_shape=[
            jax.ShapeDtypeStruct((NP, H), jnp.float32),
            jax.ShapeDtypeStruct((G, 2 * H), jnp.float32),
        ],
    )(one_eps, h_in, aggs, aggs, vn, batch3, w1p, b1p, w2, b2, v1a, v1b, b1v)


def _pre_body(pooled, h, batch3, v2w, v2b, h_in_out, vn_out):
    vn = jnp.maximum(
        jnp.dot(pooled[...], v2w[...], preferred_element_type=jnp.float32)
        + v2b[...], 0.0)
    ind = _indicator(batch3[0, 0, :])
    h_in_out[...] = h[...] + jnp.dot(ind, vn, preferred_element_type=jnp.float32)

    @pl.when(pl.program_id(0) == 0)
    def _():
        vn_out[...] = vn


def _pre_tc(pooled, h, batch3, v2w, v2b):
    full = lambda shape: pl.BlockSpec(shape, lambda i: (0,) * len(shape))
    return pl.pallas_call(
        _pre_body,
        grid=(NB,),
        in_specs=[
            full((G, 2 * H)),
            pl.BlockSpec((BR, H), lambda i: (i, 0)),
            pl.BlockSpec((1, 1, BR), lambda i: (i, 0, 0)),
            full((2 * H, H)), full((1, H)),
        ],
        out_specs=[
            pl.BlockSpec((BR, H), lambda i: (i, 0)),
            pl.BlockSpec((G, H), lambda i: (0, 0)),
        ],
        out_shape=[
            jax.ShapeDtypeStruct((NP, H), jnp.float32),
            jax.ShapeDtypeStruct((G, H), jnp.float32),
        ],
    )(pooled, h, batch3, v2w, v2b)


def _final_body(pooled, v2w, v2b, pw1, pb1, pw2, pb2, out):
    vn = jnp.maximum(
        jnp.dot(pooled[...], v2w[...], preferred_element_type=jnp.float32)
        + v2b[...], 0.0)
    p1 = jnp.maximum(
        jnp.dot(vn, pw1[...], preferred_element_type=jnp.float32) + pb1[...], 0.0)
    p2 = jnp.dot(p1, pw2[...], preferred_element_type=jnp.float32) + pb2[...]
    out[...] = jnp.clip(p2, 0.0, 50.0)


def _final_tc(pooled, v2w, v2b, pw1, pb1, pw2, pb2):
    return pl.pallas_call(
        _final_body,
        out_shape=jax.ShapeDtypeStruct((G, 1), jnp.float32),
    )(pooled, v2w, v2b, pw1, pb1, pw2, pb2)


def _fold_bn(w, b, g, bb):
    return w * g[None, :], (b * g + bb)[None, :]


def kernel(x, edge_index, edge_attr, batch, params):
    f32, i32 = jnp.float32, jnp.int32
    xi = x.astype(i32) + 100 * jnp.arange(F_ATOM, dtype=i32)[None, :]
    xi = jnp.pad(xi, ((0, NP - N), (0, 0)))
    paix = xi.T.reshape(F_ATOM, NP // AC, AC).transpose(1, 0, 2)
    atab = params['atom_emb'].reshape(F_ATOM * 100, H)

    src = jnp.concatenate(
        [edge_index[0].astype(i32), jnp.zeros((EPAD - E,), i32)])
    dst = jnp.concatenate(
        [edge_index[1].astype(i32), jnp.full((EPAD - E,), NP - 1, i32)])
    ea3 = edge_attr.astype(i32)
    ea = jnp.concatenate(
        [ea3[:, 0] * 100 + ea3[:, 1] * 10 + ea3[:, 2],
         jnp.zeros((EPAD - E,), i32)])
    pidx = jnp.stack([a.reshape(GCH, EC) for a in (src, ea, dst)], axis=1)
    zrows = jnp.zeros((STRIPE, H), f32)

    batch_p = jnp.concatenate(
        [batch.astype(i32), jnp.full((NP - N,), G, i32)])
    batch3 = batch_p.reshape(NB, 1, BR)

    atom_sc, edge_sc = _sc_kernels()
    h0 = atom_sc(atab, paix)

    layers = params['layers']
    etab_s, v2w_s, v2b_s = [], [], []
    w1p_s, b1p_s, w2_s, b2_s, v1a_s, v1b_s, b1v_s, eps_s = ([] for _ in range(8))
    for l, lp in enumerate(layers):
        etab_s.append((lp['bond_emb'][0][:, None, None, :]
                       + lp['bond_emb'][1][None, :, None, :]
                       + lp['bond_emb'][2][None, None, :, :]).reshape(1000, H))
        if l == 0:
            v2w_s.append(jnp.zeros((2 * H, H), f32))
            v2b_s.append(jnp.zeros((1, H), f32))
        else:
            pp = layers[l - 1]
            vw, vb = _fold_bn(pp['vn2_w'], pp['vn2_b'], pp['vn2_g'], pp['vn2_bb'])
            v2w_s.append(vw)
            v2b_s.append(vb)
        w1p, b1p = _fold_bn(lp['w1'], lp['b1'], lp['bn1_g'], lp['bn1_b'])
        v1w, b1v = _fold_bn(lp['vn1_w'], lp['vn1_b'], lp['vn1_g'], lp['vn1_bb'])
        w1p_s.append(w1p); b1p_s.append(b1p)
        w2_s.append(lp['w2']); b2_s.append(lp['b2'][None, :])
        v1a_s.append(v1w[:H]); v1b_s.append(v1w[H:]); b1v_s.append(b1v)
        eps_s.append((1.0 + lp['eps']).astype(f32).reshape(1, 1))
    xs = tuple(jnp.stack(a) for a in (
        etab_s, v2w_s, v2b_s, w1p_s, b1p_s, w2_s, b2_s, v1a_s, v1b_s,
        b1v_s, eps_s))

    def layer_step(carry, x):
        h_prev, pooled = carry
        etab, v2w, v2b, w1p, b1p, w2, b2, v1a, v1b, b1v, one_eps = x
        h_in, vn = _pre_tc(pooled, h_prev, batch3, v2w, v2b)
        aggs = edge_sc(h_in, etab, pidx, zrows)
        h_new, pooled_new = _post_tc(
            one_eps, h_in, aggs, vn, batch3, w1p, b1p, w2, b2, v1a, v1b, b1v)
        return (h_new, pooled_new), None

    (_, pooled), _ = lax.scan(
        layer_step, (h0, jnp.zeros((G, 2 * H), f32)), xs)

    lp_last = layers[-1]
    v2w, v2b = _fold_bn(lp_last['vn2_w'], lp_last['vn2_b'],
                        lp_last['vn2_g'], lp_last['vn2_bb'])
    return _final_tc(pooled, v2w, v2b,
                     params['pred_w1'], params['pred_b1'][None, :],
                     params['pred_w2'], params['pred_b2'][None, :])


# speedup vs baseline: 1.1829x; 1.0032x over previous
import functools

import jax
import jax.numpy as jnp
from jax import lax
from jax.experimental import pallas as pl
from jax.experimental.pallas import tpu as pltpu
from jax.experimental.pallas import tpu_sc as plsc

N = 10000
NP = 10240
E = 320000
EPAD = 327680
H = 128
G = 128
F_ATOM = 9
NC, NS, LANES = 2, 16, 16
NW = NC * NS
EPT = EPAD // NW
EC = 80
GCH = EPAD // EC
KCH0 = 200
KCH1 = 56
NPT = NP // NW
AC = 80
NCHUNK_A = NPT // AC
STRIPE = NP // NS
BR = 512
NB = NP // BR

def _vec_loop_add_relu(dst, src, rows):
    def _row(r, carry):
        for c in range(H // LANES):
            sl = pl.ds(c * LANES, LANES)
            dst[r, sl] = jnp.maximum(dst[r, sl] + src[r, sl], 0.0)
        return carry
    lax.fori_loop(0, rows, _row, 0)


def _vec_loop_add(dst, src, rows):
    def _row(r, carry):
        for c in range(H // LANES):
            sl = pl.ds(c * LANES, LANES)
            dst[r, sl] = dst[r, sl] + src[r, sl]
        return carry
    lax.fori_loop(0, rows, _row, 0)


def _atom_body(atab_hbm, paix_hbm, out_hbm, pidxa, acc, tmp, asem):
    cid = lax.axis_index("c")
    sid = lax.axis_index("s")

    def _chunk(g0):
        def body(k, carry):
            g = g0 + k
            pltpu.sync_copy(paix_hbm.at[g], pidxa)
            for f in range(F_ATOM):
                pltpu.async_copy(atab_hbm.at[pidxa.at[f]], tmp.at[f], asem)
            for f in range(F_ATOM):
                pltpu.make_async_copy(
                    atab_hbm.at[pl.ds(0, AC)], tmp.at[f], asem).wait()

            def _row(r, c2):
                for c in range(H // LANES):
                    sl = pl.ds(c * LANES, LANES)
                    v = tmp[0, r, sl]
                    for f in range(1, F_ATOM):
                        v = v + tmp[f, r, sl]
                    acc[r, sl] = v
                return c2

            lax.fori_loop(0, AC, _row, 0)
            pltpu.sync_copy(acc, out_hbm.at[pl.ds(g * AC, AC)])
            return carry
        return body

    @pl.when(cid == 0)
    def _():
        lax.fori_loop(0, 6, _chunk(sid * 6), 0)

    @pl.when(cid == 1)
    def _():
        lax.fori_loop(0, 2, _chunk(16 * 6 + sid * 2), 0)


def _edge_body(h_hbm, etab_hbm, pidx_hbm, zrows_hbm, out_hbm,
               pidx, dstc, hs, es, agg_sh,
               isem0, isem1, isem2, isem3, gsem0, gsem1, ssem0, ssem1):
    cid = lax.axis_index("c")
    sid = lax.axis_index("s")
    isems = (isem0, isem1, isem2, isem3)
    gsems, ssems = (gsem0, gsem1), (ssem0, ssem1)

    def issue_idx(g, q):
        pltpu.async_copy(pidx_hbm.at[g], pidx.at[q], isems[q])

    def wait_idx(q):
        pltpu.make_async_copy(pidx_hbm.at[0], pidx.at[q], isems[q]).wait()

    def issue_gathers(b, q):
        pltpu.async_copy(h_hbm.at[pidx.at[q, 0]], hs.at[b], gsems[b])
        pltpu.async_copy(etab_hbm.at[pidx.at[q, 1]], es.at[b], gsems[b])

    def wait_gathers(b):
        pltpu.make_async_copy(h_hbm.at[pl.ds(0, EC)], hs.at[b], gsems[b]).wait()
        pltpu.make_async_copy(etab_hbm.at[pl.ds(0, EC)], es.at[b], gsems[b]).wait()

    def copy_dst(b, q):
        for c in range(EC // LANES):
            sl = pl.ds(c * LANES, LANES)
            dstc[b, sl] = pidx[q, 2, sl]

    def compute(b):
        def _row(i, carry):
            r = i * 2
            for rr in (r, r + 1):
                for c in range(H // LANES):
                    sl = pl.ds(c * LANES, LANES)
                    hs[b, rr, sl] = jnp.maximum(hs[b, rr, sl] + es[b, rr, sl],
                                                0.0)
            return carry
        lax.fori_loop(0, EC // 2, _row, 0)

    def issue_scatter(b):
        pltpu.async_copy(hs.at[b], agg_sh.at[dstc.at[b]], ssems[b], add=True)

    def wait_scatter(b):
        pltpu.make_async_copy(hs.at[b], agg_sh.at[pl.ds(0, EC)], ssems[b]).wait()

    pltpu.sync_copy(zrows_hbm.at[pl.ds(0, EC)], hs.at[0])
    for k in range(STRIPE // EC):
        pltpu.sync_copy(hs.at[0], agg_sh.at[pl.ds(sid * STRIPE + k * EC, EC)])
    plsc.subcore_barrier()

    def _step(j, b, q, pf_idx, pf_g):
        wait_gathers(b)
        copy_dst(b, q)
        if pf_idx:
            issue_idx(cbase + j + 4, q)
        compute(b)
        issue_scatter(b)
        if pf_g:
            wait_idx((q + 2) % 4)
            wait_scatter(b)
            issue_gathers(b, (q + 2) % 4)

    def run_edges(nch):
        for q in range(4):
            issue_idx(cbase + q, q)
        wait_idx(0)
        issue_gathers(0, 0)
        wait_idx(1)
        issue_gathers(1, 1)
        _step(0, 0, 0, True, True)
        _step(1, 1, 1, True, True)

        def _mid(it, carry):
            for d in range(4):
                j = 2 + it * 4 + d
                _step(j, (2 + d) % 2, (2 + d) % 4, True, True)
            return carry

        lax.fori_loop(0, (nch - 8) // 4, _mid, 0)
        for j in range(nch - 6, nch):
            _step(j, j % 2, j % 4, j < nch - 4, j < nch - 2)
        wait_scatter(0)
        wait_scatter(1)

    cbase = jnp.where(cid == 0, sid * KCH0, 16 * KCH0 + sid * KCH1)

    @pl.when(cid == 0)
    def _():
        run_edges(KCH0)

    @pl.when(cid == 1)
    def _():
        run_edges(KCH1)

    plsc.subcore_barrier()

    for k in range(STRIPE // EC):
        b = k % 2
        r0 = sid * STRIPE + k * EC
        if k >= 2:
            pltpu.make_async_copy(
                hs.at[b], out_hbm.at[pl.ds(0, EC)], ssems[b]).wait()
        pltpu.sync_copy(agg_sh.at[pl.ds(r0, EC)], hs.at[b])
        pltpu.async_copy(hs.at[b], out_hbm.at[pl.ds(cid * NP + r0, EC)],
                         ssems[b])
    for b in range(2):
        pltpu.make_async_copy(
            hs.at[b], out_hbm.at[pl.ds(0, EC)], ssems[b]).wait()


@functools.cache
def _sc_kernels():
    mesh = plsc.VectorSubcoreMesh(core_axis_name="c", subcore_axis_name="s")
    atom = pl.kernel(
        _atom_body,
        out_type=jax.ShapeDtypeStruct((NP, H), jnp.float32),
        mesh=mesh,
        scratch_types=[
            pltpu.VMEM((F_ATOM, AC), jnp.int32),
            pltpu.VMEM((AC, H), jnp.float32),
            pltpu.VMEM((F_ATOM, AC, H), jnp.float32),
            pltpu.SemaphoreType.DMA,
        ],
    )
    edge = pl.kernel(
        _edge_body,
        out_type=jax.ShapeDtypeStruct((2 * NP, H), jnp.float32),
        mesh=mesh,
        scratch_types=[
            pltpu.VMEM((4, 3, EC), jnp.int32),
            pltpu.VMEM((2, EC), jnp.int32),
            pltpu.VMEM((2, EC, H), jnp.float32),
            pltpu.VMEM((2, EC, H), jnp.float32),
            pltpu.VMEM_SHARED((NP, H), jnp.float32),
        ] + [pltpu.SemaphoreType.DMA] * 8,
    )
    return atom, edge


def _indicator(b):
    return (b[:, None] == lax.broadcasted_iota(jnp.int32, (1, G), 1)
            ).astype(jnp.float32)


def _post_body(one_eps, h_in, agg_a, agg_b, vn, batch3,
               w1p, b1p, w2, b2, v1a, v1b, b1v, h_out, pooled):
    he = h_in[...]
    z = one_eps[0, 0] * he + (agg_a[...] + agg_b[...])
    z = jnp.maximum(
        jnp.dot(z, w1p[...], preferred_element_type=jnp.float32) + b1p[...], 0.0)
    hn = jnp.maximum(
        jnp.dot(z, w2[...], preferred_element_type=jnp.float32) + b2[...], 0.0)
    h_out[...] = hn
    ind = _indicator(batch3[0, 0, :])
    vnb = jnp.dot(ind, vn[...], preferred_element_type=jnp.float32)
    vt = jnp.maximum(
        jnp.dot(vnb, v1a[...], preferred_element_type=jnp.float32)
        + jnp.dot(hn, v1b[...], preferred_element_type=jnp.float32)
        + b1v[...], 0.0)
    part = lax.dot_general(ind, vt, (((0,), (0,)), ((), ())),
                           preferred_element_type=jnp.float32)

    @pl.when(pl.program_id(0) == 0)
    def _():
        pooled[...] = jnp.zeros_like(pooled)

    pooled[...] += part


def _post_tc(one_eps, h_in, aggs, vn, batch3, w1p, b1p, w2, b2, v1a, v1b, b1v):
    full = lambda shape: pl.BlockSpec(shape, lambda i: (0,) * len(shape))
    return pl.pallas_call(
        _post_body,
        grid=(NB,),
        in_specs=[
            full((1, 1)),
            pl.BlockSpec((BR, H), lambda i: (i, 0)),
            pl.BlockSpec((BR, H), lambda i: (i, 0)),
            pl.BlockSpec((BR, H), lambda i: (i + NB, 0)),
            full((G, H)),
            pl.BlockSpec((1, 1, BR), lambda i: (i, 0, 0)),
            full((H, 2 * H)), full((1, 2 * H)),
            full((2 * H, H)), full((1, H)),
            full((H, 2 * H)), full((H, 2 * H)), full((1, 2 * H)),
        ],
        out_specs=[
            pl.BlockSpec((BR, H), lambda i: (i, 0)),
            pl.BlockSpec((G, 2 * H), lambda i: (0, 0)),
        ],
        out_shape=[
            jax.ShapeDtypeStruct((NP, H), jnp.float32),
            jax.ShapeDtypeStruct((G, 2 * H), jnp.float32),
        ],
    )(one_eps, h_in, aggs, aggs, vn, batch3, w1p, b1p, w2, b2, v1a, v1b, b1v)


def _pre_body(pooled, h, batch3, v2w, v2b, h_in_out, vn_out):
    vn = jnp.maximum(
        jnp.dot(pooled[...], v2w[...], preferred_element_type=jnp.float32)
        + v2b[...], 0.0)
    ind = _indicator(batch3[0, 0, :])
    h_in_out[...] = h[...] + jnp.dot(ind, vn, preferred_element_type=jnp.float32)

    @pl.when(pl.program_id(0) == 0)
    def _():
        vn_out[...] = vn


def _pre_tc(pooled, h, batch3, v2w, v2b):
    full = lambda shape: pl.BlockSpec(shape, lambda i: (0,) * len(shape))
    return pl.pallas_call(
        _pre_body,
        grid=(NB,),
        in_specs=[
            full((G, 2 * H)),
            pl.BlockSpec((BR, H), lambda i: (i, 0)),
            pl.BlockSpec((1, 1, BR), lambda i: (i, 0, 0)),
            full((2 * H, H)), full((1, H)),
        ],
        out_specs=[
            pl.BlockSpec((BR, H), lambda i: (i, 0)),
            pl.BlockSpec((G, H), lambda i: (0, 0)),
        ],
        out_shape=[
            jax.ShapeDtypeStruct((NP, H), jnp.float32),
            jax.ShapeDtypeStruct((G, H), jnp.float32),
        ],
    )(pooled, h, batch3, v2w, v2b)


def _final_body(pooled, v2w, v2b, pw1, pb1, pw2, pb2, out):
    vn = jnp.maximum(
        jnp.dot(pooled[...], v2w[...], preferred_element_type=jnp.float32)
        + v2b[...], 0.0)
    p1 = jnp.maximum(
        jnp.dot(vn, pw1[...], preferred_element_type=jnp.float32) + pb1[...], 0.0)
    p2 = jnp.dot(p1, pw2[...], preferred_element_type=jnp.float32) + pb2[...]
    out[...] = jnp.clip(p2, 0.0, 50.0)


def _final_tc(pooled, v2w, v2b, pw1, pb1, pw2, pb2):
    return pl.pallas_call(
        _final_body,
        out_shape=jax.ShapeDtypeStruct((G, 1), jnp.float32),
    )(pooled, v2w, v2b, pw1, pb1, pw2, pb2)


def _fold_bn(w, b, g, bb):
    return w * g[None, :], (b * g + bb)[None, :]


def kernel(x, edge_index, edge_attr, batch, params):
    f32, i32 = jnp.float32, jnp.int32
    xi = x.astype(i32) + 100 * jnp.arange(F_ATOM, dtype=i32)[None, :]
    xi = jnp.pad(xi, ((0, NP - N), (0, 0)))
    paix = xi.T.reshape(F_ATOM, NP // AC, AC).transpose(1, 0, 2)
    atab = params['atom_emb'].reshape(F_ATOM * 100, H)

    src = jnp.concatenate(
        [edge_index[0].astype(i32), jnp.zeros((EPAD - E,), i32)])
    dst = jnp.concatenate(
        [edge_index[1].astype(i32), jnp.full((EPAD - E,), NP - 1, i32)])
    ea3 = edge_attr.astype(i32)
    ea = jnp.concatenate(
        [ea3[:, 0] * 100 + ea3[:, 1] * 10 + ea3[:, 2],
         jnp.zeros((EPAD - E,), i32)])
    pidx = jnp.stack([a.reshape(GCH, EC) for a in (src, ea, dst)], axis=1)
    zrows = jnp.zeros((STRIPE, H), f32)

    batch_p = jnp.concatenate(
        [batch.astype(i32), jnp.full((NP - N,), G, i32)])
    batch3 = batch_p.reshape(NB, 1, BR)

    atom_sc, edge_sc = _sc_kernels()
    h0 = atom_sc(atab, paix)

    layers = params['layers']
    etab_s, v2w_s, v2b_s = [], [], []
    w1p_s, b1p_s, w2_s, b2_s, v1a_s, v1b_s, b1v_s, eps_s = ([] for _ in range(8))
    for l, lp in enumerate(layers):
        etab_s.append((lp['bond_emb'][0][:, None, None, :]
                       + lp['bond_emb'][1][None, :, None, :]
                       + lp['bond_emb'][2][None, None, :, :]).reshape(1000, H))
        if l == 0:
            v2w_s.append(jnp.zeros((2 * H, H), f32))
            v2b_s.append(jnp.zeros((1, H), f32))
        else:
            pp = layers[l - 1]
            vw, vb = _fold_bn(pp['vn2_w'], pp['vn2_b'], pp['vn2_g'], pp['vn2_bb'])
            v2w_s.append(vw)
            v2b_s.append(vb)
        w1p, b1p = _fold_bn(lp['w1'], lp['b1'], lp['bn1_g'], lp['bn1_b'])
        v1w, b1v = _fold_bn(lp['vn1_w'], lp['vn1_b'], lp['vn1_g'], lp['vn1_bb'])
        w1p_s.append(w1p); b1p_s.append(b1p)
        w2_s.append(lp['w2']); b2_s.append(lp['b2'][None, :])
        v1a_s.append(v1w[:H]); v1b_s.append(v1w[H:]); b1v_s.append(b1v)
        eps_s.append((1.0 + lp['eps']).astype(f32).reshape(1, 1))
    xs = tuple(jnp.stack(a) for a in (
        etab_s, v2w_s, v2b_s, w1p_s, b1p_s, w2_s, b2_s, v1a_s, v1b_s,
        b1v_s, eps_s))

    def layer_step(carry, x):
        h_prev, pooled = carry
        etab, v2w, v2b, w1p, b1p, w2, b2, v1a, v1b, b1v, one_eps = x
        h_in, vn = _pre_tc(pooled, h_prev, batch3, v2w, v2b)
        aggs = edge_sc(h_in, etab, pidx, zrows)
        h_new, pooled_new = _post_tc(
            one_eps, h_in, aggs, vn, batch3, w1p, b1p, w2, b2, v1a, v1b, b1v)
        return (h_new, pooled_new), None

    (_, pooled), _ = lax.scan(
        layer_step, (h0, jnp.zeros((G, 2 * H), f32)), xs)

    lp_last = layers[-1]
    v2w, v2b = _fold_bn(lp_last['vn2_w'], lp_last['vn2_b'],
                        lp_last['vn2_g'], lp_last['vn2_bb'])
    return _final_tc(pooled, v2w, v2b,
                     params['pred_w1'], params['pred_b1'][None, :],
                     params['pred_w2'], params['pred_b2'][None, :])


# speedup vs baseline: 1.1852x; 1.0020x over previous
import functools

import jax
import jax.numpy as jnp
from jax import lax
from jax.experimental import pallas as pl
from jax.experimental.pallas import tpu as pltpu
from jax.experimental.pallas import tpu_sc as plsc

N = 10000
NP = 10240
E = 320000
EPAD = 327680
H = 128
G = 128
F_ATOM = 9
NC, NS, LANES = 2, 16, 16
NW = NC * NS
EPT = EPAD // NW
EC = 80
GCH = EPAD // EC
KCH0 = 200
KCH1 = 56
NPT = NP // NW
AC = 80
NCHUNK_A = NPT // AC
STRIPE = NP // NS
BR = 512
NB = NP // BR


def _atom_body(atab_hbm, paix_hbm, out_hbm, pidxa, acc, tmp, asem):
    cid = lax.axis_index("c")
    sid = lax.axis_index("s")

    def _chunk(g0):
        def body(k, carry):
            g = g0 + k
            pltpu.sync_copy(paix_hbm.at[g], pidxa)
            for f in range(F_ATOM):
                pltpu.async_copy(atab_hbm.at[pidxa.at[f]], tmp.at[f], asem)
            for f in range(F_ATOM):
                pltpu.make_async_copy(
                    atab_hbm.at[pl.ds(0, AC)], tmp.at[f], asem).wait()

            def _row(r, c2):
                for c in range(H // LANES):
                    sl = pl.ds(c * LANES, LANES)
                    v = tmp[0, r, sl]
                    for f in range(1, F_ATOM):
                        v = v + tmp[f, r, sl]
                    acc[r, sl] = v
                return c2

            lax.fori_loop(0, AC, _row, 0)
            pltpu.sync_copy(acc, out_hbm.at[pl.ds(g * AC, AC)])
            return carry
        return body

    @pl.when(cid == 0)
    def _():
        lax.fori_loop(0, 6, _chunk(sid * 6), 0)

    @pl.when(cid == 1)
    def _():
        lax.fori_loop(0, 2, _chunk(16 * 6 + sid * 2), 0)


def _edge_body(h_hbm, etab_hbm, pidx_hbm, zrows_hbm, out_hbm,
               pidx, dstc, hs, es, agg_sh,
               isem0, isem1, isem2, isem3, gsem0, gsem1, ssem0, ssem1):
    cid = lax.axis_index("c")
    sid = lax.axis_index("s")
    isems = (isem0, isem1, isem2, isem3)
    gsems, ssems = (gsem0, gsem1), (ssem0, ssem1)

    def issue_idx(g, q):
        pltpu.async_copy(pidx_hbm.at[g], pidx.at[q], isems[q])

    def wait_idx(q):
        pltpu.make_async_copy(pidx_hbm.at[0], pidx.at[q], isems[q]).wait()

    def issue_gathers(b, q):
        pltpu.async_copy(h_hbm.at[pidx.at[q, 0]], hs.at[b], gsems[b])
        pltpu.async_copy(etab_hbm.at[pidx.at[q, 1]], es.at[b], gsems[b])

    def wait_gathers(b):
        pltpu.make_async_copy(h_hbm.at[pl.ds(0, EC)], hs.at[b], gsems[b]).wait()
        pltpu.make_async_copy(etab_hbm.at[pl.ds(0, EC)], es.at[b], gsems[b]).wait()

    def copy_dst(b, q):
        for c in range(EC // LANES):
            sl = pl.ds(c * LANES, LANES)
            dstc[b, sl] = pidx[q, 2, sl]

    def compute(b):
        def _row(i, carry):
            r = i * 2
            for rr in (r, r + 1):
                for c in range(H // LANES):
                    sl = pl.ds(c * LANES, LANES)
                    hs[b, rr, sl] = jnp.maximum(hs[b, rr, sl] + es[b, rr, sl],
                                                0.0)
            return carry
        lax.fori_loop(0, EC // 2, _row, 0)

    def issue_scatter(b):
        pltpu.async_copy(hs.at[b], agg_sh.at[dstc.at[b]], ssems[b], add=True)

    def wait_scatter(b):
        pltpu.make_async_copy(hs.at[b], agg_sh.at[pl.ds(0, EC)], ssems[b]).wait()

    pltpu.sync_copy(zrows_hbm.at[pl.ds(0, EC)], hs.at[0])
    for k in range(STRIPE // EC):
        pltpu.sync_copy(hs.at[0], agg_sh.at[pl.ds(sid * STRIPE + k * EC, EC)])
    plsc.subcore_barrier()

    def _step(j, b, q, pf_idx, pf_g):
        wait_gathers(b)
        copy_dst(b, q)
        if pf_idx:
            issue_idx(cbase + j + 4, q)
        compute(b)
        issue_scatter(b)
        if pf_g:
            wait_idx((q + 2) % 4)
            wait_scatter(b)
            issue_gathers(b, (q + 2) % 4)

    def run_edges(nch):
        for q in range(4):
            issue_idx(cbase + q, q)
        wait_idx(0)
        issue_gathers(0, 0)
        wait_idx(1)
        issue_gathers(1, 1)
        _step(0, 0, 0, True, True)
        _step(1, 1, 1, True, True)

        def _mid(it, carry):
            for d in range(4):
                j = 2 + it * 4 + d
                _step(j, (2 + d) % 2, (2 + d) % 4, True, True)
            return carry

        lax.fori_loop(0, (nch - 8) // 4, _mid, 0)
        for j in range(nch - 6, nch):
            _step(j, j % 2, j % 4, j < nch - 4, j < nch - 2)
        wait_scatter(0)
        wait_scatter(1)

    cbase = jnp.where(cid == 0, sid * KCH0, 16 * KCH0 + sid * KCH1)

    @pl.when(cid == 0)
    def _():
        run_edges(KCH0)

    @pl.when(cid == 1)
    def _():
        run_edges(KCH1)

    plsc.subcore_barrier()

    for k in range(STRIPE // EC):
        b = k % 2
        r0 = sid * STRIPE + k * EC
        if k >= 2:
            pltpu.make_async_copy(
                hs.at[b], out_hbm.at[pl.ds(0, EC)], ssems[b]).wait()
        pltpu.sync_copy(agg_sh.at[pl.ds(r0, EC)], hs.at[b])
        pltpu.async_copy(hs.at[b], out_hbm.at[pl.ds(cid * NP + r0, EC)],
                         ssems[b])
    for b in range(2):
        pltpu.make_async_copy(
            hs.at[b], out_hbm.at[pl.ds(0, EC)], ssems[b]).wait()


@functools.cache
def _sc_kernels():
    mesh = plsc.VectorSubcoreMesh(core_axis_name="c", subcore_axis_name="s")
    atom = pl.kernel(
        _atom_body,
        out_type=jax.ShapeDtypeStruct((NP, H), jnp.float32),
        mesh=mesh,
        scratch_types=[
            pltpu.VMEM((F_ATOM, AC), jnp.int32),
            pltpu.VMEM((AC, H), jnp.float32),
            pltpu.VMEM((F_ATOM, AC, H), jnp.float32),
            pltpu.SemaphoreType.DMA,
        ],
    )
    edge = pl.kernel(
        _edge_body,
        out_type=jax.ShapeDtypeStruct((2 * NP, H), jnp.float32),
        mesh=mesh,
        scratch_types=[
            pltpu.VMEM((4, 3, EC), jnp.int32),
            pltpu.VMEM((2, EC), jnp.int32),
            pltpu.VMEM((2, EC, H), jnp.float32),
            pltpu.VMEM((2, EC, H), jnp.float32),
            pltpu.VMEM_SHARED((NP, H), jnp.float32),
        ] + [pltpu.SemaphoreType.DMA] * 8,
    )
    return atom, edge


def _indicator(b):
    return (b[:, None] == lax.broadcasted_iota(jnp.int32, (1, G), 1)
            ).astype(jnp.float32)


def _post_body(one_eps, h_in, agg_a, agg_b, vn, batch3,
               w1p, b1p, w2, b2, v1a, v1b, b1v, h_out, pooled):
    he = h_in[...]
    z = one_eps[0, 0] * he + (agg_a[...] + agg_b[...])
    z = jnp.maximum(
        jnp.dot(z, w1p[...], preferred_element_type=jnp.float32) + b1p[...], 0.0)
    hn = jnp.maximum(
        jnp.dot(z, w2[...], preferred_element_type=jnp.float32) + b2[...], 0.0)
    h_out[...] = hn
    ind = _indicator(batch3[0, 0, :])
    vnb = jnp.dot(ind, vn[...], preferred_element_type=jnp.float32)
    vt = jnp.maximum(
        jnp.dot(vnb, v1a[...], preferred_element_type=jnp.float32)
        + jnp.dot(hn, v1b[...], preferred_element_type=jnp.float32)
        + b1v[...], 0.0)
    part = lax.dot_general(ind, vt, (((0,), (0,)), ((), ())),
                           preferred_element_type=jnp.float32)

    @pl.when(pl.program_id(0) == 0)
    def _():
        pooled[...] = jnp.zeros_like(pooled)

    pooled[...] += part


def _post_tc(one_eps, h_in, aggs, vn, batch3, w1p, b1p, w2, b2, v1a, v1b, b1v):
    full = lambda shape: pl.BlockSpec(shape, lambda i: (0,) * len(shape))
    return pl.pallas_call(
        _post_body,
        grid=(NB,),
        in_specs=[
            full((1, 1)),
            pl.BlockSpec((BR, H), lambda i: (i, 0)),
            pl.BlockSpec((BR, H), lambda i: (i, 0)),
            pl.BlockSpec((BR, H), lambda i: (i + NB, 0)),
            full((G, H)),
            pl.BlockSpec((1, 1, BR), lambda i: (i, 0, 0)),
            full((H, 2 * H)), full((1, 2 * H)),
            full((2 * H, H)), full((1, H)),
            full((H, 2 * H)), full((H, 2 * H)), full((1, 2 * H)),
        ],
        out_specs=[
            pl.BlockSpec((BR, H), lambda i: (i, 0)),
            pl.BlockSpec((G, 2 * H), lambda i: (0, 0)),
        ],
        out_shape=[
            jax.ShapeDtypeStruct((NP, H), jnp.float32),
            jax.ShapeDtypeStruct((G, 2 * H), jnp.float32),
        ],
    )(one_eps, h_in, aggs, aggs, vn, batch3, w1p, b1p, w2, b2, v1a, v1b, b1v)


def _pre_body(pooled, h, batch3, v2w, v2b, h_in_out, vn_out):
    vn = jnp.maximum(
        jnp.dot(pooled[...], v2w[...], preferred_element_type=jnp.float32)
        + v2b[...], 0.0)
    ind = _indicator(batch3[0, 0, :])
    h_in_out[...] = h[...] + jnp.dot(ind, vn, preferred_element_type=jnp.float32)

    @pl.when(pl.program_id(0) == 0)
    def _():
        vn_out[...] = vn


def _pre_tc(pooled, h, batch3, v2w, v2b):
    full = lambda shape: pl.BlockSpec(shape, lambda i: (0,) * len(shape))
    return pl.pallas_call(
        _pre_body,
        grid=(NB,),
        in_specs=[
            full((G, 2 * H)),
            pl.BlockSpec((BR, H), lambda i: (i, 0)),
            pl.BlockSpec((1, 1, BR), lambda i: (i, 0, 0)),
            full((2 * H, H)), full((1, H)),
        ],
        out_specs=[
            pl.BlockSpec((BR, H), lambda i: (i, 0)),
            pl.BlockSpec((G, H), lambda i: (0, 0)),
        ],
        out_shape=[
            jax.ShapeDtypeStruct((NP, H), jnp.float32),
            jax.ShapeDtypeStruct((G, H), jnp.float32),
        ],
    )(pooled, h, batch3, v2w, v2b)


def _final_body(pooled, v2w, v2b, pw1, pb1, pw2, pb2, out):
    vn = jnp.maximum(
        jnp.dot(pooled[...], v2w[...], preferred_element_type=jnp.float32)
        + v2b[...], 0.0)
    p1 = jnp.maximum(
        jnp.dot(vn, pw1[...], preferred_element_type=jnp.float32) + pb1[...], 0.0)
    p2 = jnp.dot(p1, pw2[...], preferred_element_type=jnp.float32) + pb2[...]
    out[...] = jnp.clip(p2, 0.0, 50.0)


def _final_tc(pooled, v2w, v2b, pw1, pb1, pw2, pb2):
    return pl.pallas_call(
        _final_body,
        out_shape=jax.ShapeDtypeStruct((G, 1), jnp.float32),
    )(pooled, v2w, v2b, pw1, pb1, pw2, pb2)


def _fold_bn(w, b, g, bb):
    return w * g[None, :], (b * g + bb)[None, :]


def kernel(x, edge_index, edge_attr, batch, params):
    f32, i32 = jnp.float32, jnp.int32
    xi = x.astype(i32) + 100 * jnp.arange(F_ATOM, dtype=i32)[None, :]
    xi = jnp.pad(xi, ((0, NP - N), (0, 0)))
    paix = xi.T.reshape(F_ATOM, NP // AC, AC).transpose(1, 0, 2)
    atab = params['atom_emb'].reshape(F_ATOM * 100, H)

    src = jnp.concatenate(
        [edge_index[0].astype(i32), jnp.zeros((EPAD - E,), i32)])
    dst = jnp.concatenate(
        [edge_index[1].astype(i32), jnp.full((EPAD - E,), NP - 1, i32)])
    ea3 = edge_attr.astype(i32)
    ea = jnp.concatenate(
        [ea3[:, 0] * 100 + ea3[:, 1] * 10 + ea3[:, 2],
         jnp.zeros((EPAD - E,), i32)])
    pidx = jnp.stack([a.reshape(GCH, EC) for a in (src, ea, dst)], axis=1)
    zrows = jnp.zeros((STRIPE, H), f32)

    batch_p = jnp.concatenate(
        [batch.astype(i32), jnp.full((NP - N,), G, i32)])
    batch3 = batch_p.reshape(NB, 1, BR)

    atom_sc, edge_sc = _sc_kernels()
    h0 = atom_sc(atab, paix)

    layers = params['layers']
    etab_s, v2w_s, v2b_s = [], [], []
    w1p_s, b1p_s, w2_s, b2_s, v1a_s, v1b_s, b1v_s, eps_s = ([] for _ in range(8))
    for l, lp in enumerate(layers):
        etab_s.append((lp['bond_emb'][0][:, None, None, :]
                       + lp['bond_emb'][1][None, :, None, :]
                       + lp['bond_emb'][2][None, None, :, :]).reshape(1000, H))
        if l == 0:
            v2w_s.append(jnp.zeros((2 * H, H), f32))
            v2b_s.append(jnp.zeros((1, H), f32))
        else:
            pp = layers[l - 1]
            vw, vb = _fold_bn(pp['vn2_w'], pp['vn2_b'], pp['vn2_g'], pp['vn2_bb'])
            v2w_s.append(vw)
            v2b_s.append(vb)
        w1p, b1p = _fold_bn(lp['w1'], lp['b1'], lp['bn1_g'], lp['bn1_b'])
        v1w, b1v = _fold_bn(lp['vn1_w'], lp['vn1_b'], lp['vn1_g'], lp['vn1_bb'])
        w1p_s.append(w1p); b1p_s.append(b1p)
        w2_s.append(lp['w2']); b2_s.append(lp['b2'][None, :])
        v1a_s.append(v1w[:H]); v1b_s.append(v1w[H:]); b1v_s.append(b1v)
        eps_s.append((1.0 + lp['eps']).astype(f32).reshape(1, 1))
    xs = tuple(jnp.stack(a) for a in (
        etab_s, v2w_s, v2b_s, w1p_s, b1p_s, w2_s, b2_s, v1a_s, v1b_s,
        b1v_s, eps_s))

    def layer_step(carry, x):
        h_prev, pooled = carry
        etab, v2w, v2b, w1p, b1p, w2, b2, v1a, v1b, b1v, one_eps = x
        h_in, vn = _pre_tc(pooled, h_prev, batch3, v2w, v2b)
        aggs = edge_sc(h_in, etab, pidx, zrows)
        h_new, pooled_new = _post_tc(
            one_eps, h_in, aggs, vn, batch3, w1p, b1p, w2, b2, v1a, v1b, b1v)
        return (h_new, pooled_new), None

    (_, pooled), _ = lax.scan(
        layer_step, (h0, jnp.zeros((G, 2 * H), f32)), xs)

    lp_last = layers[-1]
    v2w, v2b = _fold_bn(lp_last['vn2_w'], lp_last['vn2_b'],
                        lp_last['vn2_g'], lp_last['vn2_bb'])
    return _final_tc(pooled, v2w, v2b,
                     params['pred_w1'], params['pred_b1'][None, :],
                     params['pred_w2'], params['pred_b2'][None, :])
